# baseline (speedup 1.0000x reference)
import functools

import jax
import jax.numpy as jnp
from jax import lax
from jax.experimental import pallas as pl
from jax.experimental.pallas import tpu as pltpu
from jax.experimental.pallas import tpu_sc as plsc

B = 2
S = 4096
N = B * S
K = 16
CIN = 256
MID = 128
OUT = 256
TW = 512
NK = N * K
EPS = 1e-5
BIGF = 3.0e38

_INTERPRET = False

RP = 512
RK = 512
CE = 128


def _proj_body(pts_ref, xyz_ref, Wall_ref, ball_ref, q_ref, tab_ref):
    x = pts_ref[...]
    qkv = jnp.dot(x, Wall_ref[...], preferred_element_type=jnp.float32) + ball_ref[...]
    q_ref[...] = qkv[:, 0:MID]
    tab_ref[:, 0:OUT] = qkv[:, 2 * MID:2 * MID + OUT]
    tab_ref[:, OUT:OUT + MID] = qkv[:, MID:2 * MID]
    tab_ref[:, OUT + MID:TW] = jnp.concatenate(
        [xyz_ref[...], jnp.zeros((RP, MID - 3), jnp.float32)], axis=1)


def _proj(pts, xyzf, Wall, ball):
    return pl.pallas_call(
        _proj_body,
        grid=(N // RP,),
        in_specs=[
            pl.BlockSpec((RP, CIN), lambda i: (i, 0)),
            pl.BlockSpec((RP, 3), lambda i: (i, 0)),
            pl.BlockSpec((CIN, 2 * MID + OUT), lambda i: (0, 0)),
            pl.BlockSpec((1, 2 * MID + OUT), lambda i: (0, 0)),
        ],
        out_specs=[
            pl.BlockSpec((RP, MID), lambda i: (i, 0)),
            pl.BlockSpec((RP, TW), lambda i: (i, 0)),
        ],
        out_shape=[
            jax.ShapeDtypeStruct((N, MID), jnp.float32),
            jax.ShapeDtypeStruct((N, TW), jnp.float32),
        ],
        interpret=_INTERPRET,
    )(pts, xyzf, Wall, ball)


def _knn_body(xyz_ref, xyzT_ref, idx_ref):
    i = pl.program_id(0)
    xr = xyz_ref[...]
    xcT = xyzT_ref[...]
    sqr = jnp.sum(xr * xr, axis=1, keepdims=True)
    sqc = jnp.sum(xcT * xcT, axis=0, keepdims=True)
    d = sqr + sqc - 2.0 * jnp.dot(xr, xcT, preferred_element_type=jnp.float32)
    cols = lax.broadcasted_iota(jnp.int32, (RK, S), 1)
    rows = i * RK + lax.broadcasted_iota(jnp.int32, (RK, 1), 0)
    d = jnp.where(cols == rows, BIGF, d)
    outs = [rows]
    for _ in range(K - 1):
        m = jnp.min(d, axis=1, keepdims=True)
        am = jnp.min(jnp.where(d == m, cols, jnp.int32(S)), axis=1, keepdims=True)
        outs.append(am)
        d = jnp.where(cols == am, BIGF, d)
    idx_ref[...] = jnp.concatenate(outs, axis=1)


def _knn1(xyz2, xyzT2):
    return pl.pallas_call(
        _knn_body,
        grid=(S // RK,),
        in_specs=[
            pl.BlockSpec((RK, 3), lambda i: (i, 0)),
            pl.BlockSpec((3, S), lambda i: (0, 0)),
        ],
        out_specs=pl.BlockSpec((RK, K), lambda i: (i, 0)),
        out_shape=jax.ShapeDtypeStruct((S, K), jnp.int32),
        interpret=_INTERPRET,
    )(xyz2, xyzT2)


SC_NC, SC_NS = 2, 16
NW = SC_NC * SC_NS
CH = 64
NKH = S * K


def _gather_sc(table, idxf):
    mesh = plsc.VectorSubcoreMesh(core_axis_name="c", subcore_axis_name="s")
    rpw = NKH // NW

    @functools.partial(
        pl.kernel,
        out_type=jax.ShapeDtypeStruct((NKH, TW), jnp.float32),
        mesh=mesh,
        scratch_types=[
            pltpu.VMEM((rpw,), jnp.int32),
            pltpu.VMEM((CH, TW), jnp.float32),
            pltpu.VMEM((CH, TW), jnp.float32),
            pltpu.SemaphoreType.DMA,
            pltpu.SemaphoreType.DMA,
        ],
    )
    def k(table_hbm, idx_hbm, out_hbm, idx_v, buf_a, buf_b, sem_a, sem_b):
        wid = lax.axis_index("s") * SC_NC + lax.axis_index("c")
        base = wid * rpw
        pltpu.sync_copy(idx_hbm.at[pl.ds(base, rpw)], idx_v)

        def body(i, carry):
            c0 = 2 * i * CH
            cp_a = pltpu.async_copy(
                table_hbm.at[idx_v.at[pl.ds(c0, CH)]], buf_a, sem_a)
            cp_b = pltpu.async_copy(
                table_hbm.at[idx_v.at[pl.ds(c0 + CH, CH)]], buf_b, sem_b)
            cp_a.wait()
            pltpu.sync_copy(buf_a, out_hbm.at[pl.ds(base + c0, CH)])
            cp_b.wait()
            pltpu.sync_copy(buf_b, out_hbm.at[pl.ds(base + c0 + CH, CH)])
            return carry

        lax.fori_loop(0, rpw // (2 * CH), body, 0)

    return k(table, idxf)


_gather = _gather_sc


def _stats1_body(gx_ref, xyz_ref, Wp1_ref, bp1_ref, acc_ref, p1_ref):
    nb = gx_ref[:, :, 0:3]
    ctr = xyz_ref[...][None, :, :]
    rel = (nb - ctr).reshape(K * CE, 3)
    p1 = jnp.dot(rel, Wp1_ref[...], preferred_element_type=jnp.float32) + bp1_ref[...]
    p1_ref[...] = p1.reshape(K, CE, 3)

    @pl.when(pl.program_id(0) == 0)
    def _():
        acc_ref[...] = jnp.zeros_like(acc_ref)

    acc_ref[0:1, 0:3] += jnp.sum(p1, axis=0, keepdims=True)
    acc_ref[1:2, 0:3] += jnp.sum(p1 * p1, axis=0, keepdims=True)


def _stats1(G3, xyzf, Wp1, bp1):
    return pl.pallas_call(
        _stats1_body,
        grid=(S // CE,),
        in_specs=[
            pl.BlockSpec((K, CE, MID), lambda i: (0, i, 3)),
            pl.BlockSpec((CE, 3), lambda i: (i, 0)),
            pl.BlockSpec((3, 3), lambda i: (0, 0)),
            pl.BlockSpec((1, 3), lambda i: (0, 0)),
        ],
        out_specs=[
            pl.BlockSpec((8, 128), lambda i: (0, 0)),
            pl.BlockSpec((K, CE, 3), lambda i: (0, i, 0)),
        ],
        out_shape=[
            jax.ShapeDtypeStruct((8, 128), jnp.float32),
            jax.ShapeDtypeStruct((K, S, 3), jnp.float32),
        ],
        interpret=_INTERPRET,
    )(G3, xyzf, Wp1, bp1)


def _passE_body(gk_ref, p1_ref, q_ref, sc1_ref, sh1_ref, Wp2f_ref, bp2f_ref,
                w0_ref, acc_ref):
    p1n = jnp.maximum(p1_ref[...].reshape(K * CE, 3) * sc1_ref[...] + sh1_ref[...], 0.0)
    fold = jnp.dot(p1n, Wp2f_ref[...], preferred_element_type=jnp.float32) + bp2f_ref[...]
    w0 = gk_ref[...] - q_ref[...][None, :, :] + fold.reshape(K, CE, MID)
    w0_ref[...] = w0
    w2d = w0.reshape(K * CE, MID)

    @pl.when(pl.program_id(0) == 0)
    def _():
        acc_ref[...] = jnp.zeros_like(acc_ref)

    acc_ref[0:1, :] += jnp.sum(w2d, axis=0, keepdims=True)
    acc_ref[1:2, :] += jnp.sum(w2d * w2d, axis=0, keepdims=True)


def _passE(G3, p1, q, sc1, sh1, Wp2f, bp2f):
    return pl.pallas_call(
        _passE_body,
        grid=(S // CE,),
        in_specs=[
            pl.BlockSpec((K, CE, MID), lambda i: (0, i, 2)),
            pl.BlockSpec((K, CE, 3), lambda i: (0, i, 0)),
            pl.BlockSpec((CE, MID), lambda i: (i, 0)),
            pl.BlockSpec((1, 3), lambda i: (0, 0)),
            pl.BlockSpec((1, 3), lambda i: (0, 0)),
            pl.BlockSpec((3, MID), lambda i: (0, 0)),
            pl.BlockSpec((1, MID), lambda i: (0, 0)),
        ],
        out_specs=[
            pl.BlockSpec((K, CE, MID), lambda i: (0, i, 0)),
            pl.BlockSpec((8, 128), lambda i: (0, 0)),
        ],
        out_shape=[
            jax.ShapeDtypeStruct((K, S, MID), jnp.float32),
            jax.ShapeDtypeStruct((8, 128), jnp.float32),
        ],
        interpret=_INTERPRET,
    )(G3, p1, q, sc1, sh1, Wp2f, bp2f)


def _passF_body(w0_ref, sc2_ref, sh2_ref, Ww1_ref, bw1_ref, w1_ref, acc_ref):
    w0 = w0_ref[...].reshape(K * CE, MID)
    w0n = jnp.maximum(w0 * sc2_ref[...] + sh2_ref[...], 0.0)
    w1 = jnp.dot(w0n, Ww1_ref[...], preferred_element_type=jnp.float32) + bw1_ref[...]
    w1_ref[...] = w1.reshape(K, CE, 16)

    @pl.when(pl.program_id(0) == 0)
    def _():
        acc_ref[...] = jnp.zeros_like(acc_ref)

    acc_ref[0:1, 0:16] += jnp.sum(w1, axis=0, keepdims=True)
    acc_ref[1:2, 0:16] += jnp.sum(w1 * w1, axis=0, keepdims=True)


def _passF(w0, sc2, sh2, Ww1, bw1):
    return pl.pallas_call(
        _passF_body,
        grid=(S // CE,),
        in_specs=[
            pl.BlockSpec((K, CE, MID), lambda i: (0, i, 0)),
            pl.BlockSpec((1, MID), lambda i: (0, 0)),
            pl.BlockSpec((1, MID), lambda i: (0, 0)),
            pl.BlockSpec((MID, 16), lambda i: (0, 0)),
            pl.BlockSpec((1, 16), lambda i: (0, 0)),
        ],
        out_specs=[
            pl.BlockSpec((K, CE, 16), lambda i: (0, i, 0)),
            pl.BlockSpec((8, 128), lambda i: (0, 0)),
        ],
        out_shape=[
            jax.ShapeDtypeStruct((K, S, 16), jnp.float32),
            jax.ShapeDtypeStruct((8, 128), jnp.float32),
        ],
        interpret=_INTERPRET,
    )(w0, sc2, sh2, Ww1, bw1)


def _passG_body(w1_ref, gv_ref, p1_ref, sc3_ref, sh3_ref, Ww2_ref, bw2_ref,
                sc1_ref, sh1_ref, Wp2_ref, bp2_ref, out_ref):
    w1 = w1_ref[...].reshape(K * CE, 16)
    w1n = jnp.maximum(w1 * sc3_ref[...] + sh3_ref[...], 0.0)
    w2 = (jnp.dot(w1n, Ww2_ref[...], preferred_element_type=jnp.float32)
          + bw2_ref[...]).reshape(K, CE, 32)
    m = jnp.max(w2, axis=0, keepdims=True)
    e = jnp.exp(w2 - m)
    sm = e / jnp.sum(e, axis=0, keepdims=True)
    p1n = jnp.maximum(p1_ref[...].reshape(K * CE, 3) * sc1_ref[...] + sh1_ref[...], 0.0)
    pr = (jnp.dot(p1n, Wp2_ref[...], preferred_element_type=jnp.float32)
          + bp2_ref[...]).reshape(K, CE, OUT)
    x2 = gv_ref[...] + pr
    wt = jnp.concatenate([sm] * 8, axis=2)
    out_ref[...] = jnp.sum(x2 * wt, axis=0)


def _passG(w1, G3, p1, sc3, sh3, Ww2, bw2, sc1, sh1, Wp2, bp2):
    return pl.pallas_call(
        _passG_body,
        grid=(S // CE,),
        in_specs=[
            pl.BlockSpec((K, CE, 16), lambda i: (0, i, 0)),
            pl.BlockSpec((K, CE, OUT), lambda i: (0, i, 0)),
            pl.BlockSpec((K, CE, 3), lambda i: (0, i, 0)),
            pl.BlockSpec((1, 16), lambda i: (0, 0)),
            pl.BlockSpec((1, 16), lambda i: (0, 0)),
            pl.BlockSpec((16, 32), lambda i: (0, 0)),
            pl.BlockSpec((1, 32), lambda i: (0, 0)),
            pl.BlockSpec((1, 3), lambda i: (0, 0)),
            pl.BlockSpec((1, 3), lambda i: (0, 0)),
            pl.BlockSpec((3, OUT), lambda i: (0, 0)),
            pl.BlockSpec((1, OUT), lambda i: (0, 0)),
        ],
        out_specs=pl.BlockSpec((CE, OUT), lambda i: (i, 0)),
        out_shape=jax.ShapeDtypeStruct((S, OUT), jnp.float32),
        interpret=_INTERPRET,
    )(w1, G3, p1, sc3, sh3, Ww2, bw2, sc1, sh1, Wp2, bp2)


def kernel(xyz, points, Wq, bq, Wk, bk, Wv, bv, Wp1, bp1, gp, bp, Wp2, bp2,
           g1, b1, Ww1, bw1, g2, b2, Ww2, bw2):
    ptsf = points.reshape(N, CIN)
    xyzf = xyz.reshape(N, 3)
    Wall = jnp.concatenate([Wq, Wk, Wv], axis=1)
    ball = jnp.concatenate([bq, bk, bv])[None, :]
    q, table = _proj(ptsf, xyzf, Wall, ball)

    G3s = []
    for b in range(B):
        xyz2 = xyz[b]
        idx = _knn1(xyz2, jnp.transpose(xyz2))
        Gb = _gather(table, jnp.transpose(idx + b * S).reshape(NKH))
        G3s.append(Gb.reshape(K, S, TW))

    cnt = jnp.float32(NK)
    accs, p1s = [], []
    for b in range(B):
        a, p1 = _stats1(G3s[b], xyzf[b * S:(b + 1) * S], Wp1, bp1[None, :])
        accs.append(a)
        p1s.append(p1)
    acc1 = accs[0] + accs[1]
    m1 = acc1[0, 0:3] / cnt
    v1 = acc1[1, 0:3] / cnt - m1 * m1
    sc1 = gp * lax.rsqrt(v1 + EPS)
    sh1 = bp - m1 * sc1

    Wp2f = Wp2[:, 0:MID] + Wp2[:, MID:OUT]
    bp2f = (bp2[0:MID] + bp2[MID:OUT])[None, :]
    w0s = []
    acc2 = None
    for b in range(B):
        w0, a = _passE(G3s[b], p1s[b], q[b * S:(b + 1) * S],
                       sc1[None], sh1[None], Wp2f, bp2f)
        w0s.append(w0)
        acc2 = a if acc2 is None else acc2 + a
    m2 = acc2[0] / cnt
    v2 = acc2[1] / cnt - m2 * m2
    sc2 = g1 * lax.rsqrt(v2 + EPS)
    sh2 = b1 - m2 * sc2

    w1s = []
    acc3 = None
    for b in range(B):
        w1, a = _passF(w0s[b], sc2[None], sh2[None], Ww1, bw1[None])
        w1s.append(w1)
        acc3 = a if acc3 is None else acc3 + a
    m3 = acc3[0, 0:16] / cnt
    v3 = acc3[1, 0:16] / cnt - m3 * m3
    sc3 = g2 * lax.rsqrt(v3 + EPS)
    sh3 = b2 - m3 * sc3

    outs = [
        _passG(w1s[b], G3s[b], p1s[b], sc3[None], sh3[None], Ww2, bw2[None],
               sc1[None], sh1[None], Wp2, bp2[None])
        for b in range(B)
    ]
    return jnp.concatenate(outs, axis=0)


# speedup vs baseline: 1.0545x; 1.0545x over previous
import functools

import jax
import jax.numpy as jnp
from jax import lax
from jax.experimental import pallas as pl
from jax.experimental.pallas import tpu as pltpu
from jax.experimental.pallas import tpu_sc as plsc

B = 2
S = 4096
N = B * S
K = 16
CIN = 256
MID = 128
OUT = 256
TWI = 256
NK = N * K
EPS = 1e-5
BIGF = 3.0e38

_INTERPRET = False

RP = 512
RK = 512
CE = 128


def _pack_bf16(a, b):
    ua = lax.bitcast_convert_type(a, jnp.uint32)
    ub = lax.bitcast_convert_type(b, jnp.uint32)
    ra = (ua + jnp.uint32(0x7FFF) + ((ua >> 16) & jnp.uint32(1))) >> 16
    rb = (ub + jnp.uint32(0x7FFF) + ((ub >> 16) & jnp.uint32(1))) >> 16
    return lax.bitcast_convert_type(ra | (rb << 16), jnp.int32)


def _unpack_lo(p):
    return lax.bitcast_convert_type(p << 16, jnp.float32)


def _unpack_hi(p):
    u = lax.bitcast_convert_type(p, jnp.uint32)
    return lax.bitcast_convert_type((u >> 16) << 16, jnp.float32)


def _proj_body(pts_ref, xyz_ref, Wall_ref, ball_ref, q_ref, tab_ref):
    x = pts_ref[...]
    qkv = jnp.dot(x, Wall_ref[...], preferred_element_type=jnp.float32) + ball_ref[...]
    q_ref[...] = qkv[:, 0:MID]
    v = qkv[:, 2 * MID:2 * MID + OUT]
    k = qkv[:, MID:2 * MID]
    tab_ref[:, 0:128] = _pack_bf16(v[:, 0:128], v[:, 128:256])
    kp = _pack_bf16(k[:, 0:64], k[:, 64:128])
    xyzb = lax.bitcast_convert_type(xyz_ref[...], jnp.int32)
    tab_ref[:, 128:256] = jnp.concatenate(
        [kp, xyzb, jnp.zeros((RP, 61), jnp.int32)], axis=1)


def _proj(pts, xyzf, Wall, ball):
    return pl.pallas_call(
        _proj_body,
        grid=(N // RP,),
        in_specs=[
            pl.BlockSpec((RP, CIN), lambda i: (i, 0)),
            pl.BlockSpec((RP, 3), lambda i: (i, 0)),
            pl.BlockSpec((CIN, 2 * MID + OUT), lambda i: (0, 0)),
            pl.BlockSpec((1, 2 * MID + OUT), lambda i: (0, 0)),
        ],
        out_specs=[
            pl.BlockSpec((RP, MID), lambda i: (i, 0)),
            pl.BlockSpec((RP, TWI), lambda i: (i, 0)),
        ],
        out_shape=[
            jax.ShapeDtypeStruct((N, MID), jnp.float32),
            jax.ShapeDtypeStruct((N, TWI), jnp.int32),
        ],
        interpret=_INTERPRET,
    )(pts, xyzf, Wall, ball)


def _knn_body(xyz_ref, xyzT_ref, idx_ref):
    i = pl.program_id(0)
    xr = xyz_ref[...]
    xcT = xyzT_ref[...]
    sqr = jnp.sum(xr * xr, axis=1, keepdims=True)
    sqc = jnp.sum(xcT * xcT, axis=0, keepdims=True)
    d = sqr + sqc - 2.0 * jnp.dot(xr, xcT, preferred_element_type=jnp.float32)
    cols = lax.broadcasted_iota(jnp.int32, (RK, S), 1)
    rows = i * RK + lax.broadcasted_iota(jnp.int32, (RK, 1), 0)
    d = jnp.where(cols == rows, BIGF, d)
    outs = [rows]
    for _ in range(K - 1):
        m = jnp.min(d, axis=1, keepdims=True)
        am = jnp.min(jnp.where(d == m, cols, jnp.int32(S)), axis=1, keepdims=True)
        outs.append(am)
        d = jnp.where(cols == am, BIGF, d)
    idx_ref[...] = jnp.concatenate(outs, axis=1)


def _knn1(xyz2, xyzT2):
    return pl.pallas_call(
        _knn_body,
        grid=(S // RK,),
        in_specs=[
            pl.BlockSpec((RK, 3), lambda i: (i, 0)),
            pl.BlockSpec((3, S), lambda i: (0, 0)),
        ],
        out_specs=pl.BlockSpec((RK, K), lambda i: (i, 0)),
        out_shape=jax.ShapeDtypeStruct((S, K), jnp.int32),
        interpret=_INTERPRET,
    )(xyz2, xyzT2)


SC_NC, SC_NS = 2, 16
NW = SC_NC * SC_NS
CH = 128
NKH = S * K


def _gather_sc(table, idxf):
    mesh = plsc.VectorSubcoreMesh(core_axis_name="c", subcore_axis_name="s")
    rpw = NKH // NW

    @functools.partial(
        pl.kernel,
        out_type=jax.ShapeDtypeStruct((NKH, TWI), jnp.int32),
        mesh=mesh,
        scratch_types=[
            pltpu.VMEM((rpw,), jnp.int32),
            pltpu.VMEM((CH, TWI), jnp.int32),
            pltpu.VMEM((CH, TWI), jnp.int32),
            pltpu.SemaphoreType.DMA,
            pltpu.SemaphoreType.DMA,
        ],
    )
    def k(table_hbm, idx_hbm, out_hbm, idx_v, buf_a, buf_b, sem_a, sem_b):
        wid = lax.axis_index("s") * SC_NC + lax.axis_index("c")
        base = wid * rpw
        pltpu.sync_copy(idx_hbm.at[pl.ds(base, rpw)], idx_v)

        def body(i, carry):
            c0 = 2 * i * CH
            cp_a = pltpu.async_copy(
                table_hbm.at[idx_v.at[pl.ds(c0, CH)]], buf_a, sem_a)
            cp_b = pltpu.async_copy(
                table_hbm.at[idx_v.at[pl.ds(c0 + CH, CH)]], buf_b, sem_b)
            cp_a.wait()
            pltpu.sync_copy(buf_a, out_hbm.at[pl.ds(base + c0, CH)])
            cp_b.wait()
            pltpu.sync_copy(buf_b, out_hbm.at[pl.ds(base + c0 + CH, CH)])
            return carry

        lax.fori_loop(0, rpw // (2 * CH), body, 0)

    return k(table, idxf)


_gather = _gather_sc


def _stats1_body(gx_ref, xyz_ref, Wp1_ref, bp1_ref, acc_ref, p1_ref):
    nb = lax.bitcast_convert_type(gx_ref[:, :, 64:67], jnp.float32)
    ctr = xyz_ref[...][:, None, :]
    rel = (nb - ctr).reshape(CE * K, 3)
    p1 = jnp.dot(rel, Wp1_ref[...], preferred_element_type=jnp.float32) + bp1_ref[...]
    p1_ref[...] = p1

    @pl.when(pl.program_id(0) == 0)
    def _():
        acc_ref[...] = jnp.zeros_like(acc_ref)

    acc_ref[0:1, 0:3] += jnp.sum(p1, axis=0, keepdims=True)
    acc_ref[1:2, 0:3] += jnp.sum(p1 * p1, axis=0, keepdims=True)


def _stats1(G3, xyzf, Wp1, bp1):
    return pl.pallas_call(
        _stats1_body,
        grid=(S // CE,),
        in_specs=[
            pl.BlockSpec((CE, K, 128), lambda i: (i, 0, 1)),
            pl.BlockSpec((CE, 3), lambda i: (i, 0)),
            pl.BlockSpec((3, 3), lambda i: (0, 0)),
            pl.BlockSpec((1, 3), lambda i: (0, 0)),
        ],
        out_specs=[
            pl.BlockSpec((8, 128), lambda i: (0, 0)),
            pl.BlockSpec((CE * K, 3), lambda i: (i, 0)),
        ],
        out_shape=[
            jax.ShapeDtypeStruct((8, 128), jnp.float32),
            jax.ShapeDtypeStruct((NKH, 3), jnp.float32),
        ],
        interpret=_INTERPRET,
    )(G3, xyzf, Wp1, bp1)


def _passE_body(gk_ref, p1_ref, q_ref, sc1_ref, sh1_ref, Wp2f_ref, bp2f_ref,
                w0_ref, acc_ref):
    p1n = jnp.maximum(p1_ref[...] * sc1_ref[...] + sh1_ref[...], 0.0)
    fold = jnp.dot(p1n, Wp2f_ref[...], preferred_element_type=jnp.float32) + bp2f_ref[...]
    kp = gk_ref[:, :, 0:64]
    gk = jnp.concatenate([_unpack_lo(kp), _unpack_hi(kp)], axis=2)
    w0 = gk - q_ref[...][:, None, :] + fold.reshape(CE, K, MID)
    w0_ref[...] = w0
    w2d = w0.reshape(CE * K, MID)

    @pl.when(pl.program_id(0) == 0)
    def _():
        acc_ref[...] = jnp.zeros_like(acc_ref)

    acc_ref[0:1, :] += jnp.sum(w2d, axis=0, keepdims=True)
    acc_ref[1:2, :] += jnp.sum(w2d * w2d, axis=0, keepdims=True)


def _passE(G3, p1, q, sc1, sh1, Wp2f, bp2f):
    return pl.pallas_call(
        _passE_body,
        grid=(S // CE,),
        in_specs=[
            pl.BlockSpec((CE, K, 128), lambda i: (i, 0, 1)),
            pl.BlockSpec((CE * K, 3), lambda i: (i, 0)),
            pl.BlockSpec((CE, MID), lambda i: (i, 0)),
            pl.BlockSpec((1, 3), lambda i: (0, 0)),
            pl.BlockSpec((1, 3), lambda i: (0, 0)),
            pl.BlockSpec((3, MID), lambda i: (0, 0)),
            pl.BlockSpec((1, MID), lambda i: (0, 0)),
        ],
        out_specs=[
            pl.BlockSpec((CE, K, MID), lambda i: (i, 0, 0)),
            pl.BlockSpec((8, 128), lambda i: (0, 0)),
        ],
        out_shape=[
            jax.ShapeDtypeStruct((S, K, MID), jnp.float32),
            jax.ShapeDtypeStruct((8, 128), jnp.float32),
        ],
        interpret=_INTERPRET,
    )(G3, p1, q, sc1, sh1, Wp2f, bp2f)


def _passF_body(w0_ref, sc2_ref, sh2_ref, Ww1_ref, bw1_ref, w1_ref, acc_ref):
    w0 = w0_ref[...].reshape(CE * K, MID)
    w0n = jnp.maximum(w0 * sc2_ref[...] + sh2_ref[...], 0.0)
    w1 = jnp.dot(w0n, Ww1_ref[...], preferred_element_type=jnp.float32) + bw1_ref[...]
    w1_ref[...] = w1.reshape(CE, K, 16)

    @pl.when(pl.program_id(0) == 0)
    def _():
        acc_ref[...] = jnp.zeros_like(acc_ref)

    acc_ref[0:1, 0:16] += jnp.sum(w1, axis=0, keepdims=True)
    acc_ref[1:2, 0:16] += jnp.sum(w1 * w1, axis=0, keepdims=True)


def _passF(w0, sc2, sh2, Ww1, bw1):
    return pl.pallas_call(
        _passF_body,
        grid=(S // CE,),
        in_specs=[
            pl.BlockSpec((CE, K, MID), lambda i: (i, 0, 0)),
            pl.BlockSpec((1, MID), lambda i: (0, 0)),
            pl.BlockSpec((1, MID), lambda i: (0, 0)),
            pl.BlockSpec((MID, 16), lambda i: (0, 0)),
            pl.BlockSpec((1, 16), lambda i: (0, 0)),
        ],
        out_specs=[
            pl.BlockSpec((CE, K, 16), lambda i: (i, 0, 0)),
            pl.BlockSpec((8, 128), lambda i: (0, 0)),
        ],
        out_shape=[
            jax.ShapeDtypeStruct((S, K, 16), jnp.float32),
            jax.ShapeDtypeStruct((8, 128), jnp.float32),
        ],
        interpret=_INTERPRET,
    )(w0, sc2, sh2, Ww1, bw1)


def _passG_body(w1_ref, gv_ref, p1_ref, sc3_ref, sh3_ref, Ww2_ref, bw2_ref,
                sc1_ref, sh1_ref, Wp2_ref, bp2_ref, out_ref):
    w1 = w1_ref[...].reshape(CE * K, 16)
    w1n = jnp.maximum(w1 * sc3_ref[...] + sh3_ref[...], 0.0)
    w2 = (jnp.dot(w1n, Ww2_ref[...], preferred_element_type=jnp.float32)
          + bw2_ref[...]).reshape(CE, K, 32)
    m = jnp.max(w2, axis=1, keepdims=True)
    e = jnp.exp(w2 - m)
    sm = e / jnp.sum(e, axis=1, keepdims=True)
    p1n = jnp.maximum(p1_ref[...] * sc1_ref[...] + sh1_ref[...], 0.0)
    pr = (jnp.dot(p1n, Wp2_ref[...], preferred_element_type=jnp.float32)
          + bp2_ref[...]).reshape(CE, K, OUT)
    vp = gv_ref[...]
    gv = jnp.concatenate([_unpack_lo(vp), _unpack_hi(vp)], axis=2)
    x2 = gv + pr
    wt = jnp.concatenate([sm] * 8, axis=2)
    out_ref[...] = jnp.sum(x2 * wt, axis=1)


def _passG(w1, G3, p1, sc3, sh3, Ww2, bw2, sc1, sh1, Wp2, bp2):
    return pl.pallas_call(
        _passG_body,
        grid=(S // CE,),
        in_specs=[
            pl.BlockSpec((CE, K, 16), lambda i: (i, 0, 0)),
            pl.BlockSpec((CE, K, 128), lambda i: (i, 0, 0)),
            pl.BlockSpec((CE * K, 3), lambda i: (i, 0)),
            pl.BlockSpec((1, 16), lambda i: (0, 0)),
            pl.BlockSpec((1, 16), lambda i: (0, 0)),
            pl.BlockSpec((16, 32), lambda i: (0, 0)),
            pl.BlockSpec((1, 32), lambda i: (0, 0)),
            pl.BlockSpec((1, 3), lambda i: (0, 0)),
            pl.BlockSpec((1, 3), lambda i: (0, 0)),
            pl.BlockSpec((3, OUT), lambda i: (0, 0)),
            pl.BlockSpec((1, OUT), lambda i: (0, 0)),
        ],
        out_specs=pl.BlockSpec((CE, OUT), lambda i: (i, 0)),
        out_shape=jax.ShapeDtypeStruct((S, OUT), jnp.float32),
        interpret=_INTERPRET,
    )(w1, G3, p1, sc3, sh3, Ww2, bw2, sc1, sh1, Wp2, bp2)


def kernel(xyz, points, Wq, bq, Wk, bk, Wv, bv, Wp1, bp1, gp, bp, Wp2, bp2,
           g1, b1, Ww1, bw1, g2, b2, Ww2, bw2):
    ptsf = points.reshape(N, CIN)
    xyzf = xyz.reshape(N, 3)
    Wall = jnp.concatenate([Wq, Wk, Wv], axis=1)
    ball = jnp.concatenate([bq, bk, bv])[None, :]
    q, table = _proj(ptsf, xyzf, Wall, ball)

    G3s = []
    for b in range(B):
        xyz2 = xyz[b]
        idx = _knn1(xyz2, jnp.transpose(xyz2))
        Gb = _gather(table, (idx + b * S).reshape(NKH))
        G3s.append(Gb.reshape(S, K, TWI))

    cnt = jnp.float32(NK)
    accs, p1s = [], []
    for b in range(B):
        a, p1 = _stats1(G3s[b], xyzf[b * S:(b + 1) * S], Wp1, bp1[None, :])
        accs.append(a)
        p1s.append(p1)
    acc1 = accs[0] + accs[1]
    m1 = acc1[0, 0:3] / cnt
    v1 = acc1[1, 0:3] / cnt - m1 * m1
    sc1 = gp * lax.rsqrt(v1 + EPS)
    sh1 = bp - m1 * sc1

    Wp2f = Wp2[:, 0:MID] + Wp2[:, MID:OUT]
    bp2f = (bp2[0:MID] + bp2[MID:OUT])[None, :]
    w0s = []
    acc2 = None
    for b in range(B):
        w0, a = _passE(G3s[b], p1s[b], q[b * S:(b + 1) * S],
                       sc1[None], sh1[None], Wp2f, bp2f)
        w0s.append(w0)
        acc2 = a if acc2 is None else acc2 + a
    m2 = acc2[0] / cnt
    v2 = acc2[1] / cnt - m2 * m2
    sc2 = g1 * lax.rsqrt(v2 + EPS)
    sh2 = b1 - m2 * sc2

    w1s = []
    acc3 = None
    for b in range(B):
        w1, a = _passF(w0s[b], sc2[None], sh2[None], Ww1, bw1[None])
        w1s.append(w1)
        acc3 = a if acc3 is None else acc3 + a
    m3 = acc3[0, 0:16] / cnt
    v3 = acc3[1, 0:16] / cnt - m3 * m3
    sc3 = g2 * lax.rsqrt(v3 + EPS)
    sh3 = b2 - m3 * sc3

    outs = [
        _passG(w1s[b], G3s[b], p1s[b], sc3[None], sh3[None], Ww2, bw2[None],
               sc1[None], sh1[None], Wp2, bp2[None])
        for b in range(B)
    ]
    return jnp.concatenate(outs, axis=0)


# speedup vs baseline: 1.1231x; 1.0651x over previous
import functools

import jax
import jax.numpy as jnp
from jax import lax
from jax.experimental import pallas as pl
from jax.experimental.pallas import tpu as pltpu
from jax.experimental.pallas import tpu_sc as plsc

B = 2
S = 4096
N = B * S
K = 16
CIN = 256
MID = 128
OUT = 256
TWI = 256
NK = N * K
EPS = 1e-5
BIGF = 3.0e38

_INTERPRET = False

RP = 512
RK = 512
CE = 128


def _pack_bf16(a, b):
    ua = lax.bitcast_convert_type(a, jnp.uint32)
    ub = lax.bitcast_convert_type(b, jnp.uint32)
    ra = (ua + jnp.uint32(0x7FFF) + ((ua >> 16) & jnp.uint32(1))) >> 16
    rb = (ub + jnp.uint32(0x7FFF) + ((ub >> 16) & jnp.uint32(1))) >> 16
    return lax.bitcast_convert_type(ra | (rb << 16), jnp.int32)


def _unpack_lo(p):
    return lax.bitcast_convert_type(p << 16, jnp.float32)


def _unpack_hi(p):
    u = lax.bitcast_convert_type(p, jnp.uint32)
    return lax.bitcast_convert_type((u >> 16) << 16, jnp.float32)


def _proj_body(pts_ref, xyz_ref, Wall_ref, ball_ref, q_ref, tab_ref):
    x = pts_ref[...]
    qkv = jnp.dot(x, Wall_ref[...], preferred_element_type=jnp.float32) + ball_ref[...]
    q_ref[...] = qkv[:, 0:MID]
    v = qkv[:, 2 * MID:2 * MID + OUT]
    k = qkv[:, MID:2 * MID]
    tab_ref[:, 0:128] = _pack_bf16(v[:, 0:128], v[:, 128:256])
    kp = _pack_bf16(k[:, 0:64], k[:, 64:128])
    xyzb = lax.bitcast_convert_type(xyz_ref[...], jnp.int32)
    tab_ref[:, 128:256] = jnp.concatenate(
        [kp, xyzb, jnp.zeros((RP, 61), jnp.int32)], axis=1)


def _proj(pts, xyzf, Wall, ball):
    return pl.pallas_call(
        _proj_body,
        grid=(N // RP,),
        in_specs=[
            pl.BlockSpec((RP, CIN), lambda i: (i, 0)),
            pl.BlockSpec((RP, 3), lambda i: (i, 0)),
            pl.BlockSpec((CIN, 2 * MID + OUT), lambda i: (0, 0)),
            pl.BlockSpec((1, 2 * MID + OUT), lambda i: (0, 0)),
        ],
        out_specs=[
            pl.BlockSpec((RP, MID), lambda i: (i, 0)),
            pl.BlockSpec((RP, TWI), lambda i: (i, 0)),
        ],
        out_shape=[
            jax.ShapeDtypeStruct((N, MID), jnp.float32),
            jax.ShapeDtypeStruct((N, TWI), jnp.int32),
        ],
        interpret=_INTERPRET,
    )(pts, xyzf, Wall, ball)


def _knn_body(xyz_ref, xyzT_ref, idx_ref):
    i = pl.program_id(0)
    xr = xyz_ref[...]
    xcT = xyzT_ref[...]
    sqr = jnp.sum(xr * xr, axis=1, keepdims=True)
    sqc = jnp.sum(xcT * xcT, axis=0, keepdims=True)
    d = sqr + sqc - 2.0 * jnp.dot(xr, xcT, preferred_element_type=jnp.float32)
    cols = lax.broadcasted_iota(jnp.int32, (RK, S), 1)
    rows = i * RK + lax.broadcasted_iota(jnp.int32, (RK, 1), 0)
    d = jnp.where(cols == rows, BIGF, d)
    outs = [rows]
    for _ in range(K - 1):
        am = jnp.argmin(d, axis=1).astype(jnp.int32)[:, None]
        outs.append(am)
        d = jnp.where(cols == am, BIGF, d)
    idx_ref[...] = jnp.concatenate(outs, axis=1)


def _knn1(xyz2, xyzT2):
    return pl.pallas_call(
        _knn_body,
        grid=(S // RK,),
        in_specs=[
            pl.BlockSpec((RK, 3), lambda i: (i, 0)),
            pl.BlockSpec((3, S), lambda i: (0, 0)),
        ],
        out_specs=pl.BlockSpec((RK, K), lambda i: (i, 0)),
        out_shape=jax.ShapeDtypeStruct((S, K), jnp.int32),
        interpret=_INTERPRET,
    )(xyz2, xyzT2)


SC_NC, SC_NS = 2, 16
NW = SC_NC * SC_NS
CH = 128
NKH = S * K


def _gather_sc(table, idxf):
    mesh = plsc.VectorSubcoreMesh(core_axis_name="c", subcore_axis_name="s")
    rpw = NKH // NW

    @functools.partial(
        pl.kernel,
        out_type=jax.ShapeDtypeStruct((NKH, TWI), jnp.int32),
        mesh=mesh,
        scratch_types=[
            pltpu.VMEM((rpw,), jnp.int32),
            pltpu.VMEM((CH, TWI), jnp.int32),
            pltpu.VMEM((CH, TWI), jnp.int32),
            pltpu.SemaphoreType.DMA,
            pltpu.SemaphoreType.DMA,
        ],
    )
    def k(table_hbm, idx_hbm, out_hbm, idx_v, buf_a, buf_b, sem_a, sem_b):
        wid = lax.axis_index("s") * SC_NC + lax.axis_index("c")
        base = wid * rpw
        pltpu.sync_copy(idx_hbm.at[pl.ds(base, rpw)], idx_v)

        def body(i, carry):
            c0 = 2 * i * CH
            cp_a = pltpu.async_copy(
                table_hbm.at[idx_v.at[pl.ds(c0, CH)]], buf_a, sem_a)
            cp_b = pltpu.async_copy(
                table_hbm.at[idx_v.at[pl.ds(c0 + CH, CH)]], buf_b, sem_b)
            cp_a.wait()
            pltpu.sync_copy(buf_a, out_hbm.at[pl.ds(base + c0, CH)])
            cp_b.wait()
            pltpu.sync_copy(buf_b, out_hbm.at[pl.ds(base + c0 + CH, CH)])
            return carry

        lax.fori_loop(0, rpw // (2 * CH), body, 0)

    return k(table, idxf)


_gather = _gather_sc


def _stats1_body(gx_ref, xyz_ref, Wp1_ref, bp1_ref, acc_ref, p1_ref):
    nb = lax.bitcast_convert_type(gx_ref[:, :, 64:67], jnp.float32)
    ctr = xyz_ref[...][:, None, :]
    rel = (nb - ctr).reshape(CE * K, 3)
    p1 = jnp.dot(rel, Wp1_ref[...], preferred_element_type=jnp.float32) + bp1_ref[...]
    p1_ref[...] = p1

    @pl.when(pl.program_id(0) == 0)
    def _():
        acc_ref[...] = jnp.zeros_like(acc_ref)

    acc_ref[0:1, 0:3] += jnp.sum(p1, axis=0, keepdims=True)
    acc_ref[1:2, 0:3] += jnp.sum(p1 * p1, axis=0, keepdims=True)


def _stats1(G3, xyzf, Wp1, bp1):
    return pl.pallas_call(
        _stats1_body,
        grid=(S // CE,),
        in_specs=[
            pl.BlockSpec((CE, K, 128), lambda i: (i, 0, 1)),
            pl.BlockSpec((CE, 3), lambda i: (i, 0)),
            pl.BlockSpec((3, 3), lambda i: (0, 0)),
            pl.BlockSpec((1, 3), lambda i: (0, 0)),
        ],
        out_specs=[
            pl.BlockSpec((8, 128), lambda i: (0, 0)),
            pl.BlockSpec((CE * K, 3), lambda i: (i, 0)),
        ],
        out_shape=[
            jax.ShapeDtypeStruct((8, 128), jnp.float32),
            jax.ShapeDtypeStruct((NKH, 3), jnp.float32),
        ],
        interpret=_INTERPRET,
    )(G3, xyzf, Wp1, bp1)


def _passE_body(gk_ref, p1_ref, q_ref, sc1_ref, sh1_ref, Wp2f_ref, bp2f_ref,
                w0_ref, acc_ref):
    p1n = jnp.maximum(p1_ref[...] * sc1_ref[...] + sh1_ref[...], 0.0)
    fold = jnp.dot(p1n, Wp2f_ref[...], preferred_element_type=jnp.float32) + bp2f_ref[...]
    kp = gk_ref[:, :, 0:64]
    gk = jnp.concatenate([_unpack_lo(kp), _unpack_hi(kp)], axis=2)
    w0 = gk - q_ref[...][:, None, :] + fold.reshape(CE, K, MID)
    w0_ref[...] = w0
    w2d = w0.reshape(CE * K, MID)

    @pl.when(pl.program_id(0) == 0)
    def _():
        acc_ref[...] = jnp.zeros_like(acc_ref)

    acc_ref[0:1, :] += jnp.sum(w2d, axis=0, keepdims=True)
    acc_ref[1:2, :] += jnp.sum(w2d * w2d, axis=0, keepdims=True)


def _passE(G3, p1, q, sc1, sh1, Wp2f, bp2f):
    return pl.pallas_call(
        _passE_body,
        grid=(S // CE,),
        in_specs=[
            pl.BlockSpec((CE, K, 128), lambda i: (i, 0, 1)),
            pl.BlockSpec((CE * K, 3), lambda i: (i, 0)),
            pl.BlockSpec((CE, MID), lambda i: (i, 0)),
            pl.BlockSpec((1, 3), lambda i: (0, 0)),
            pl.BlockSpec((1, 3), lambda i: (0, 0)),
            pl.BlockSpec((3, MID), lambda i: (0, 0)),
            pl.BlockSpec((1, MID), lambda i: (0, 0)),
        ],
        out_specs=[
            pl.BlockSpec((CE, K, MID), lambda i: (i, 0, 0)),
            pl.BlockSpec((8, 128), lambda i: (0, 0)),
        ],
        out_shape=[
            jax.ShapeDtypeStruct((S, K, MID), jnp.float32),
            jax.ShapeDtypeStruct((8, 128), jnp.float32),
        ],
        interpret=_INTERPRET,
    )(G3, p1, q, sc1, sh1, Wp2f, bp2f)


def _passF_body(w0_ref, sc2_ref, sh2_ref, Ww1_ref, bw1_ref, w1_ref, acc_ref):
    w0 = w0_ref[...].reshape(CE * K, MID)
    w0n = jnp.maximum(w0 * sc2_ref[...] + sh2_ref[...], 0.0)
    w1 = jnp.dot(w0n, Ww1_ref[...], preferred_element_type=jnp.float32) + bw1_ref[...]
    w1_ref[...] = w1.reshape(CE, K, 16)

    @pl.when(pl.program_id(0) == 0)
    def _():
        acc_ref[...] = jnp.zeros_like(acc_ref)

    acc_ref[0:1, 0:16] += jnp.sum(w1, axis=0, keepdims=True)
    acc_ref[1:2, 0:16] += jnp.sum(w1 * w1, axis=0, keepdims=True)


def _passF(w0, sc2, sh2, Ww1, bw1):
    return pl.pallas_call(
        _passF_body,
        grid=(S // CE,),
        in_specs=[
            pl.BlockSpec((CE, K, MID), lambda i: (i, 0, 0)),
            pl.BlockSpec((1, MID), lambda i: (0, 0)),
            pl.BlockSpec((1, MID), lambda i: (0, 0)),
            pl.BlockSpec((MID, 16), lambda i: (0, 0)),
            pl.BlockSpec((1, 16), lambda i: (0, 0)),
        ],
        out_specs=[
            pl.BlockSpec((CE, K, 16), lambda i: (i, 0, 0)),
            pl.BlockSpec((8, 128), lambda i: (0, 0)),
        ],
        out_shape=[
            jax.ShapeDtypeStruct((S, K, 16), jnp.float32),
            jax.ShapeDtypeStruct((8, 128), jnp.float32),
        ],
        interpret=_INTERPRET,
    )(w0, sc2, sh2, Ww1, bw1)


def _passG_body(w1_ref, gv_ref, p1_ref, sc3_ref, sh3_ref, Ww2_ref, bw2_ref,
                sc1_ref, sh1_ref, Wp2_ref, bp2_ref, out_ref):
    w1 = w1_ref[...].reshape(CE * K, 16)
    w1n = jnp.maximum(w1 * sc3_ref[...] + sh3_ref[...], 0.0)
    w2 = (jnp.dot(w1n, Ww2_ref[...], preferred_element_type=jnp.float32)
          + bw2_ref[...]).reshape(CE, K, 32)
    m = jnp.max(w2, axis=1, keepdims=True)
    e = jnp.exp(w2 - m)
    sm = e / jnp.sum(e, axis=1, keepdims=True)
    p1n = jnp.maximum(p1_ref[...] * sc1_ref[...] + sh1_ref[...], 0.0)
    pr = (jnp.dot(p1n, Wp2_ref[...], preferred_element_type=jnp.float32)
          + bp2_ref[...]).reshape(CE, K, OUT)
    vp = gv_ref[...]
    gv = jnp.concatenate([_unpack_lo(vp), _unpack_hi(vp)], axis=2)
    x2 = gv + pr
    wt = jnp.concatenate([sm] * 8, axis=2)
    out_ref[...] = jnp.sum(x2 * wt, axis=1)


def _passG(w1, G3, p1, sc3, sh3, Ww2, bw2, sc1, sh1, Wp2, bp2):
    return pl.pallas_call(
        _passG_body,
        grid=(S // CE,),
        in_specs=[
            pl.BlockSpec((CE, K, 16), lambda i: (i, 0, 0)),
            pl.BlockSpec((CE, K, 128), lambda i: (i, 0, 0)),
            pl.BlockSpec((CE * K, 3), lambda i: (i, 0)),
            pl.BlockSpec((1, 16), lambda i: (0, 0)),
            pl.BlockSpec((1, 16), lambda i: (0, 0)),
            pl.BlockSpec((16, 32), lambda i: (0, 0)),
            pl.BlockSpec((1, 32), lambda i: (0, 0)),
            pl.BlockSpec((1, 3), lambda i: (0, 0)),
            pl.BlockSpec((1, 3), lambda i: (0, 0)),
            pl.BlockSpec((3, OUT), lambda i: (0, 0)),
            pl.BlockSpec((1, OUT), lambda i: (0, 0)),
        ],
        out_specs=pl.BlockSpec((CE, OUT), lambda i: (i, 0)),
        out_shape=jax.ShapeDtypeStruct((S, OUT), jnp.float32),
        interpret=_INTERPRET,
    )(w1, G3, p1, sc3, sh3, Ww2, bw2, sc1, sh1, Wp2, bp2)


def kernel(xyz, points, Wq, bq, Wk, bk, Wv, bv, Wp1, bp1, gp, bp, Wp2, bp2,
           g1, b1, Ww1, bw1, g2, b2, Ww2, bw2):
    ptsf = points.reshape(N, CIN)
    xyzf = xyz.reshape(N, 3)
    Wall = jnp.concatenate([Wq, Wk, Wv], axis=1)
    ball = jnp.concatenate([bq, bk, bv])[None, :]
    q, table = _proj(ptsf, xyzf, Wall, ball)

    G3s = []
    for b in range(B):
        xyz2 = xyz[b]
        idx = _knn1(xyz2, jnp.transpose(xyz2))
        Gb = _gather(table, (idx + b * S).reshape(NKH))
        G3s.append(Gb.reshape(S, K, TWI))

    cnt = jnp.float32(NK)
    accs, p1s = [], []
    for b in range(B):
        a, p1 = _stats1(G3s[b], xyzf[b * S:(b + 1) * S], Wp1, bp1[None, :])
        accs.append(a)
        p1s.append(p1)
    acc1 = accs[0] + accs[1]
    m1 = acc1[0, 0:3] / cnt
    v1 = acc1[1, 0:3] / cnt - m1 * m1
    sc1 = gp * lax.rsqrt(v1 + EPS)
    sh1 = bp - m1 * sc1

    Wp2f = Wp2[:, 0:MID] + Wp2[:, MID:OUT]
    bp2f = (bp2[0:MID] + bp2[MID:OUT])[None, :]
    w0s = []
    acc2 = None
    for b in range(B):
        w0, a = _passE(G3s[b], p1s[b], q[b * S:(b + 1) * S],
                       sc1[None], sh1[None], Wp2f, bp2f)
        w0s.append(w0)
        acc2 = a if acc2 is None else acc2 + a
    m2 = acc2[0] / cnt
    v2 = acc2[1] / cnt - m2 * m2
    sc2 = g1 * lax.rsqrt(v2 + EPS)
    sh2 = b1 - m2 * sc2

    w1s = []
    acc3 = None
    for b in range(B):
        w1, a = _passF(w0s[b], sc2[None], sh2[None], Ww1, bw1[None])
        w1s.append(w1)
        acc3 = a if acc3 is None else acc3 + a
    m3 = acc3[0, 0:16] / cnt
    v3 = acc3[1, 0:16] / cnt - m3 * m3
    sc3 = g2 * lax.rsqrt(v3 + EPS)
    sh3 = b2 - m3 * sc3

    outs = [
        _passG(w1s[b], G3s[b], p1s[b], sc3[None], sh3[None], Ww2, bw2[None],
               sc1[None], sh1[None], Wp2, bp2[None])
        for b in range(B)
    ]
    return jnp.concatenate(outs, axis=0)


# speedup vs baseline: 1.1963x; 1.0652x over previous
import functools

import jax
import jax.numpy as jnp
from jax import lax
from jax.experimental import pallas as pl
from jax.experimental.pallas import tpu as pltpu
from jax.experimental.pallas import tpu_sc as plsc

B = 2
S = 4096
N = B * S
K = 16
CIN = 256
MID = 128
OUT = 256
TWI = 256
NK = N * K
EPS = 1e-5
BIGF = 3.0e38

_INTERPRET = False

RP = 512
RK = 512
CE = 256


def _pack_bf16(a, b):
    ua = lax.bitcast_convert_type(a, jnp.uint32)
    ub = lax.bitcast_convert_type(b, jnp.uint32)
    ra = (ua + jnp.uint32(0x7FFF) + ((ua >> 16) & jnp.uint32(1))) >> 16
    rb = (ub + jnp.uint32(0x7FFF) + ((ub >> 16) & jnp.uint32(1))) >> 16
    return lax.bitcast_convert_type(ra | (rb << 16), jnp.int32)


def _unpack_lo(p):
    return lax.bitcast_convert_type(p << 16, jnp.float32)


def _unpack_hi(p):
    u = lax.bitcast_convert_type(p, jnp.uint32)
    return lax.bitcast_convert_type((u >> 16) << 16, jnp.float32)


def _proj_body(pts_ref, xyz_ref, Wall_ref, ball_ref, q_ref, tab_ref):
    x = pts_ref[...]
    qkv = jnp.dot(x, Wall_ref[...], preferred_element_type=jnp.float32) + ball_ref[...]
    q_ref[...] = qkv[:, 0:MID]
    v = qkv[:, 2 * MID:2 * MID + OUT]
    k = qkv[:, MID:2 * MID]
    tab_ref[:, 0:128] = _pack_bf16(v[:, 0:128], v[:, 128:256])
    kp = _pack_bf16(k[:, 0:64], k[:, 64:128])
    xyzb = lax.bitcast_convert_type(xyz_ref[...], jnp.int32)
    tab_ref[:, 128:256] = jnp.concatenate(
        [kp, xyzb, jnp.zeros((RP, 61), jnp.int32)], axis=1)


def _proj(pts, xyzf, Wall, ball):
    return pl.pallas_call(
        _proj_body,
        grid=(N // RP,),
        in_specs=[
            pl.BlockSpec((RP, CIN), lambda i: (i, 0)),
            pl.BlockSpec((RP, 3), lambda i: (i, 0)),
            pl.BlockSpec((CIN, 2 * MID + OUT), lambda i: (0, 0)),
            pl.BlockSpec((1, 2 * MID + OUT), lambda i: (0, 0)),
        ],
        out_specs=[
            pl.BlockSpec((RP, MID), lambda i: (i, 0)),
            pl.BlockSpec((RP, TWI), lambda i: (i, 0)),
        ],
        out_shape=[
            jax.ShapeDtypeStruct((N, MID), jnp.float32),
            jax.ShapeDtypeStruct((N, TWI), jnp.int32),
        ],
        interpret=_INTERPRET,
    )(pts, xyzf, Wall, ball)


def _knn_body(xyz_ref, xyzT_ref, idx_ref):
    i = pl.program_id(0)
    xr = xyz_ref[...]
    xcT = xyzT_ref[...]
    sqr = jnp.sum(xr * xr, axis=1, keepdims=True)
    sqc = jnp.sum(xcT * xcT, axis=0, keepdims=True)
    d = sqr + sqc - 2.0 * jnp.dot(xr, xcT, preferred_element_type=jnp.float32)
    cols = lax.broadcasted_iota(jnp.int32, (RK, S), 1)
    rows = i * RK + lax.broadcasted_iota(jnp.int32, (RK, 1), 0)
    d = jnp.where(cols == rows, BIGF, d)
    outs = [rows]
    for _ in range(K - 1):
        am = jnp.argmin(d, axis=1).astype(jnp.int32)[:, None]
        outs.append(am)
        d = jnp.where(cols == am, BIGF, d)
    idx_ref[...] = jnp.concatenate(outs, axis=1)


def _knn1(xyz2, xyzT2):
    return pl.pallas_call(
        _knn_body,
        grid=(S // RK,),
        in_specs=[
            pl.BlockSpec((RK, 3), lambda i: (i, 0)),
            pl.BlockSpec((3, S), lambda i: (0, 0)),
        ],
        out_specs=pl.BlockSpec((RK, K), lambda i: (i, 0)),
        out_shape=jax.ShapeDtypeStruct((S, K), jnp.int32),
        interpret=_INTERPRET,
    )(xyz2, xyzT2)


SC_NC, SC_NS = 2, 16
NW = SC_NC * SC_NS
CH = 128
NKH = S * K


def _gather_sc(table, idxf):
    mesh = plsc.VectorSubcoreMesh(core_axis_name="c", subcore_axis_name="s")
    rpw = NKH // NW

    @functools.partial(
        pl.kernel,
        out_type=jax.ShapeDtypeStruct((NKH, TWI), jnp.int32),
        mesh=mesh,
        scratch_types=[
            pltpu.VMEM((rpw,), jnp.int32),
            pltpu.VMEM((CH, TWI), jnp.int32),
            pltpu.VMEM((CH, TWI), jnp.int32),
            pltpu.SemaphoreType.DMA,
            pltpu.SemaphoreType.DMA,
        ],
    )
    def k(table_hbm, idx_hbm, out_hbm, idx_v, buf_a, buf_b, sem_a, sem_b):
        wid = lax.axis_index("s") * SC_NC + lax.axis_index("c")
        base = wid * rpw
        pltpu.sync_copy(idx_hbm.at[pl.ds(base, rpw)], idx_v)

        def body(i, carry):
            c0 = 2 * i * CH
            cp_a = pltpu.async_copy(
                table_hbm.at[idx_v.at[pl.ds(c0, CH)]], buf_a, sem_a)
            cp_b = pltpu.async_copy(
                table_hbm.at[idx_v.at[pl.ds(c0 + CH, CH)]], buf_b, sem_b)
            cp_a.wait()
            pltpu.sync_copy(buf_a, out_hbm.at[pl.ds(base + c0, CH)])
            cp_b.wait()
            pltpu.sync_copy(buf_b, out_hbm.at[pl.ds(base + c0 + CH, CH)])
            return carry

        lax.fori_loop(0, rpw // (2 * CH), body, 0)

    return k(table, idxf)


_gather = _gather_sc


def _stats1_body(gx_ref, xyz_ref, Wp1_ref, bp1_ref, acc_ref, p1_ref):
    nb = lax.bitcast_convert_type(gx_ref[:, :, 64:67], jnp.float32)
    ctr = xyz_ref[...][:, None, :]
    rel = (nb - ctr).reshape(CE * K, 3)
    p1 = jnp.dot(rel, Wp1_ref[...], preferred_element_type=jnp.float32) + bp1_ref[...]
    p1_ref[...] = p1

    @pl.when(pl.program_id(0) == 0)
    def _():
        acc_ref[...] = jnp.zeros_like(acc_ref)

    acc_ref[0:1, 0:3] += jnp.sum(p1, axis=0, keepdims=True)
    acc_ref[1:2, 0:3] += jnp.sum(p1 * p1, axis=0, keepdims=True)


def _stats1(G3, xyzf, Wp1, bp1):
    return pl.pallas_call(
        _stats1_body,
        grid=(S // CE,),
        in_specs=[
            pl.BlockSpec((CE, K, 128), lambda i: (i, 0, 1)),
            pl.BlockSpec((CE, 3), lambda i: (i, 0)),
            pl.BlockSpec((3, 3), lambda i: (0, 0)),
            pl.BlockSpec((1, 3), lambda i: (0, 0)),
        ],
        out_specs=[
            pl.BlockSpec((8, 128), lambda i: (0, 0)),
            pl.BlockSpec((CE * K, 3), lambda i: (i, 0)),
        ],
        out_shape=[
            jax.ShapeDtypeStruct((8, 128), jnp.float32),
            jax.ShapeDtypeStruct((NKH, 3), jnp.float32),
        ],
        interpret=_INTERPRET,
    )(G3, xyzf, Wp1, bp1)


def _passE_body(gk_ref, p1_ref, q_ref, sc1_ref, sh1_ref, Wp2f_ref, bp2f_ref,
                w0_ref, acc_ref):
    p1n = jnp.maximum(p1_ref[...] * sc1_ref[...] + sh1_ref[...], 0.0)
    fold = jnp.dot(p1n, Wp2f_ref[...], preferred_element_type=jnp.float32) + bp2f_ref[...]
    kp = gk_ref[:, :, 0:64]
    gk = jnp.concatenate([_unpack_lo(kp), _unpack_hi(kp)], axis=2)
    w0 = gk - q_ref[...][:, None, :] + fold.reshape(CE, K, MID)
    w0_ref[...] = w0
    w2d = w0.reshape(CE * K, MID)

    @pl.when(pl.program_id(0) == 0)
    def _():
        acc_ref[...] = jnp.zeros_like(acc_ref)

    acc_ref[0:1, :] += jnp.sum(w2d, axis=0, keepdims=True)
    acc_ref[1:2, :] += jnp.sum(w2d * w2d, axis=0, keepdims=True)


def _passE(G3, p1, q, sc1, sh1, Wp2f, bp2f):
    return pl.pallas_call(
        _passE_body,
        grid=(S // CE,),
        in_specs=[
            pl.BlockSpec((CE, K, 128), lambda i: (i, 0, 1)),
            pl.BlockSpec((CE * K, 3), lambda i: (i, 0)),
            pl.BlockSpec((CE, MID), lambda i: (i, 0)),
            pl.BlockSpec((1, 3), lambda i: (0, 0)),
            pl.BlockSpec((1, 3), lambda i: (0, 0)),
            pl.BlockSpec((3, MID), lambda i: (0, 0)),
            pl.BlockSpec((1, MID), lambda i: (0, 0)),
        ],
        out_specs=[
            pl.BlockSpec((CE, K, MID), lambda i: (i, 0, 0)),
            pl.BlockSpec((8, 128), lambda i: (0, 0)),
        ],
        out_shape=[
            jax.ShapeDtypeStruct((S, K, MID), jnp.float32),
            jax.ShapeDtypeStruct((8, 128), jnp.float32),
        ],
        interpret=_INTERPRET,
    )(G3, p1, q, sc1, sh1, Wp2f, bp2f)


def _passF_body(w0_ref, sc2_ref, sh2_ref, Ww1_ref, bw1_ref, w1_ref, acc_ref):
    w0 = w0_ref[...].reshape(CE * K, MID)
    w0n = jnp.maximum(w0 * sc2_ref[...] + sh2_ref[...], 0.0)
    w1 = jnp.dot(w0n, Ww1_ref[...], preferred_element_type=jnp.float32) + bw1_ref[...]
    w1_ref[...] = w1.reshape(CE, K, 16)

    @pl.when(pl.program_id(0) == 0)
    def _():
        acc_ref[...] = jnp.zeros_like(acc_ref)

    acc_ref[0:1, 0:16] += jnp.sum(w1, axis=0, keepdims=True)
    acc_ref[1:2, 0:16] += jnp.sum(w1 * w1, axis=0, keepdims=True)


def _passF(w0, sc2, sh2, Ww1, bw1):
    return pl.pallas_call(
        _passF_body,
        grid=(S // CE,),
        in_specs=[
            pl.BlockSpec((CE, K, MID), lambda i: (i, 0, 0)),
            pl.BlockSpec((1, MID), lambda i: (0, 0)),
            pl.BlockSpec((1, MID), lambda i: (0, 0)),
            pl.BlockSpec((MID, 16), lambda i: (0, 0)),
            pl.BlockSpec((1, 16), lambda i: (0, 0)),
        ],
        out_specs=[
            pl.BlockSpec((CE, K, 16), lambda i: (i, 0, 0)),
            pl.BlockSpec((8, 128), lambda i: (0, 0)),
        ],
        out_shape=[
            jax.ShapeDtypeStruct((S, K, 16), jnp.float32),
            jax.ShapeDtypeStruct((8, 128), jnp.float32),
        ],
        interpret=_INTERPRET,
    )(w0, sc2, sh2, Ww1, bw1)


def _passG_body(w1_ref, gv_ref, p1_ref, sc3_ref, sh3_ref, Ww2_ref, bw2_ref,
                sc1_ref, sh1_ref, Wp2_ref, bp2_ref, out_ref):
    w1 = w1_ref[...].reshape(CE * K, 16)
    w1n = jnp.maximum(w1 * sc3_ref[...] + sh3_ref[...], 0.0)
    w2 = (jnp.dot(w1n, Ww2_ref[...], preferred_element_type=jnp.float32)
          + bw2_ref[...]).reshape(CE, K, 32)
    m = jnp.max(w2, axis=1, keepdims=True)
    e = jnp.exp(w2 - m)
    sm = e / jnp.sum(e, axis=1, keepdims=True)
    p1n = jnp.maximum(p1_ref[...] * sc1_ref[...] + sh1_ref[...], 0.0)
    pr = (jnp.dot(p1n, Wp2_ref[...], preferred_element_type=jnp.float32)
          + bp2_ref[...]).reshape(CE, K, OUT)
    vp = gv_ref[...]
    gv = jnp.concatenate([_unpack_lo(vp), _unpack_hi(vp)], axis=2)
    x2 = gv + pr
    wt = jnp.concatenate([sm] * 8, axis=2)
    out_ref[...] = jnp.sum(x2 * wt, axis=1)


def _passG(w1, G3, p1, sc3, sh3, Ww2, bw2, sc1, sh1, Wp2, bp2):
    return pl.pallas_call(
        _passG_body,
        grid=(S // CE,),
        in_specs=[
            pl.BlockSpec((CE, K, 16), lambda i: (i, 0, 0)),
            pl.BlockSpec((CE, K, 128), lambda i: (i, 0, 0)),
            pl.BlockSpec((CE * K, 3), lambda i: (i, 0)),
            pl.BlockSpec((1, 16), lambda i: (0, 0)),
            pl.BlockSpec((1, 16), lambda i: (0, 0)),
            pl.BlockSpec((16, 32), lambda i: (0, 0)),
            pl.BlockSpec((1, 32), lambda i: (0, 0)),
            pl.BlockSpec((1, 3), lambda i: (0, 0)),
            pl.BlockSpec((1, 3), lambda i: (0, 0)),
            pl.BlockSpec((3, OUT), lambda i: (0, 0)),
            pl.BlockSpec((1, OUT), lambda i: (0, 0)),
        ],
        out_specs=pl.BlockSpec((CE, OUT), lambda i: (i, 0)),
        out_shape=jax.ShapeDtypeStruct((S, OUT), jnp.float32),
        interpret=_INTERPRET,
    )(w1, G3, p1, sc3, sh3, Ww2, bw2, sc1, sh1, Wp2, bp2)


def kernel(xyz, points, Wq, bq, Wk, bk, Wv, bv, Wp1, bp1, gp, bp, Wp2, bp2,
           g1, b1, Ww1, bw1, g2, b2, Ww2, bw2):
    ptsf = points.reshape(N, CIN)
    xyzf = xyz.reshape(N, 3)
    Wall = jnp.concatenate([Wq, Wk, Wv], axis=1)
    ball = jnp.concatenate([bq, bk, bv])[None, :]
    q, table = _proj(ptsf, xyzf, Wall, ball)

    G3s = []
    for b in range(B):
        xyz2 = xyz[b]
        idx = _knn1(xyz2, jnp.transpose(xyz2))
        Gb = _gather(table, (idx + b * S).reshape(NKH))
        G3s.append(Gb.reshape(S, K, TWI))

    cnt = jnp.float32(NK)
    accs, p1s = [], []
    for b in range(B):
        a, p1 = _stats1(G3s[b], xyzf[b * S:(b + 1) * S], Wp1, bp1[None, :])
        accs.append(a)
        p1s.append(p1)
    acc1 = accs[0] + accs[1]
    m1 = acc1[0, 0:3] / cnt
    v1 = acc1[1, 0:3] / cnt - m1 * m1
    sc1 = gp * lax.rsqrt(v1 + EPS)
    sh1 = bp - m1 * sc1

    Wp2f = Wp2[:, 0:MID] + Wp2[:, MID:OUT]
    bp2f = (bp2[0:MID] + bp2[MID:OUT])[None, :]
    w0s = []
    acc2 = None
    for b in range(B):
        w0, a = _passE(G3s[b], p1s[b], q[b * S:(b + 1) * S],
                       sc1[None], sh1[None], Wp2f, bp2f)
        w0s.append(w0)
        acc2 = a if acc2 is None else acc2 + a
    m2 = acc2[0] / cnt
    v2 = acc2[1] / cnt - m2 * m2
    sc2 = g1 * lax.rsqrt(v2 + EPS)
    sh2 = b1 - m2 * sc2

    w1s = []
    acc3 = None
    for b in range(B):
        w1, a = _passF(w0s[b], sc2[None], sh2[None], Ww1, bw1[None])
        w1s.append(w1)
        acc3 = a if acc3 is None else acc3 + a
    m3 = acc3[0, 0:16] / cnt
    v3 = acc3[1, 0:16] / cnt - m3 * m3
    sc3 = g2 * lax.rsqrt(v3 + EPS)
    sh3 = b2 - m3 * sc3

    outs = [
        _passG(w1s[b], G3s[b], p1s[b], sc3[None], sh3[None], Ww2, bw2[None],
               sc1[None], sh1[None], Wp2, bp2[None])
        for b in range(B)
    ]
    return jnp.concatenate(outs, axis=0)


# speedup vs baseline: 1.2094x; 1.0109x over previous
import functools

import jax
import jax.numpy as jnp
from jax import lax
from jax.experimental import pallas as pl
from jax.experimental.pallas import tpu as pltpu
from jax.experimental.pallas import tpu_sc as plsc

B = 2
S = 4096
N = B * S
K = 16
CIN = 256
MID = 128
OUT = 256
TWI = 256
NK = N * K
EPS = 1e-5
BIGF = 3.0e38

_INTERPRET = False

RP = 512
RK = 512
CE = 512


def _pack_bf16(a, b):
    ua = lax.bitcast_convert_type(a, jnp.uint32)
    ub = lax.bitcast_convert_type(b, jnp.uint32)
    ra = (ua + jnp.uint32(0x7FFF) + ((ua >> 16) & jnp.uint32(1))) >> 16
    rb = (ub + jnp.uint32(0x7FFF) + ((ub >> 16) & jnp.uint32(1))) >> 16
    return lax.bitcast_convert_type(ra | (rb << 16), jnp.int32)


def _unpack_lo(p):
    return lax.bitcast_convert_type(p << 16, jnp.float32)


def _unpack_hi(p):
    u = lax.bitcast_convert_type(p, jnp.uint32)
    return lax.bitcast_convert_type((u >> 16) << 16, jnp.float32)


def _proj_body(pts_ref, xyz_ref, Wall_ref, ball_ref, q_ref, tab_ref):
    x = pts_ref[...]
    qkv = jnp.dot(x, Wall_ref[...], preferred_element_type=jnp.float32) + ball_ref[...]
    q_ref[...] = qkv[:, 0:MID]
    v = qkv[:, 2 * MID:2 * MID + OUT]
    k = qkv[:, MID:2 * MID]
    tab_ref[:, 0:128] = _pack_bf16(v[:, 0:128], v[:, 128:256])
    kp = _pack_bf16(k[:, 0:64], k[:, 64:128])
    xyzb = lax.bitcast_convert_type(xyz_ref[...], jnp.int32)
    tab_ref[:, 128:256] = jnp.concatenate(
        [kp, xyzb, jnp.zeros((RP, 61), jnp.int32)], axis=1)


def _proj(pts, xyzf, Wall, ball):
    return pl.pallas_call(
        _proj_body,
        grid=(N // RP,),
        in_specs=[
            pl.BlockSpec((RP, CIN), lambda i: (i, 0)),
            pl.BlockSpec((RP, 3), lambda i: (i, 0)),
            pl.BlockSpec((CIN, 2 * MID + OUT), lambda i: (0, 0)),
            pl.BlockSpec((1, 2 * MID + OUT), lambda i: (0, 0)),
        ],
        out_specs=[
            pl.BlockSpec((RP, MID), lambda i: (i, 0)),
            pl.BlockSpec((RP, TWI), lambda i: (i, 0)),
        ],
        out_shape=[
            jax.ShapeDtypeStruct((N, MID), jnp.float32),
            jax.ShapeDtypeStruct((N, TWI), jnp.int32),
        ],
        interpret=_INTERPRET,
    )(pts, xyzf, Wall, ball)


def _knn_body(xyz_ref, xyzT_ref, idx_ref):
    i = pl.program_id(0)
    xr = xyz_ref[...]
    xcT = xyzT_ref[...]
    sqr = jnp.sum(xr * xr, axis=1, keepdims=True)
    sqc = jnp.sum(xcT * xcT, axis=0, keepdims=True)
    d = sqr + sqc - 2.0 * jnp.dot(xr, xcT, preferred_element_type=jnp.float32)
    cols = lax.broadcasted_iota(jnp.int32, (RK, S), 1)
    rows = i * RK + lax.broadcasted_iota(jnp.int32, (RK, 1), 0)
    d = jnp.where(cols == rows, BIGF, d)
    outs = [rows]
    for _ in range(K - 1):
        am = jnp.argmin(d, axis=1).astype(jnp.int32)[:, None]
        outs.append(am)
        d = jnp.where(cols == am, BIGF, d)
    idx_ref[...] = jnp.concatenate(outs, axis=1)


def _knn1(xyz2, xyzT2):
    return pl.pallas_call(
        _knn_body,
        grid=(S // RK,),
        in_specs=[
            pl.BlockSpec((RK, 3), lambda i: (i, 0)),
            pl.BlockSpec((3, S), lambda i: (0, 0)),
        ],
        out_specs=pl.BlockSpec((RK, K), lambda i: (i, 0)),
        out_shape=jax.ShapeDtypeStruct((S, K), jnp.int32),
        interpret=_INTERPRET,
    )(xyz2, xyzT2)


SC_NC, SC_NS = 2, 16
NW = SC_NC * SC_NS
CH = 128
NKH = S * K


def _gather_sc(table, idxf):
    mesh = plsc.VectorSubcoreMesh(core_axis_name="c", subcore_axis_name="s")
    rpw = NKH // NW

    @functools.partial(
        pl.kernel,
        out_type=jax.ShapeDtypeStruct((NKH, TWI), jnp.int32),
        mesh=mesh,
        scratch_types=[
            pltpu.VMEM((rpw,), jnp.int32),
            pltpu.VMEM((CH, TWI), jnp.int32),
            pltpu.VMEM((CH, TWI), jnp.int32),
            pltpu.SemaphoreType.DMA,
            pltpu.SemaphoreType.DMA,
        ],
    )
    def k(table_hbm, idx_hbm, out_hbm, idx_v, buf_a, buf_b, sem_a, sem_b):
        wid = lax.axis_index("s") * SC_NC + lax.axis_index("c")
        base = wid * rpw
        pltpu.sync_copy(idx_hbm.at[pl.ds(base, rpw)], idx_v)

        def body(i, carry):
            c0 = 2 * i * CH
            cp_a = pltpu.async_copy(
                table_hbm.at[idx_v.at[pl.ds(c0, CH)]], buf_a, sem_a)
            cp_b = pltpu.async_copy(
                table_hbm.at[idx_v.at[pl.ds(c0 + CH, CH)]], buf_b, sem_b)
            cp_a.wait()
            pltpu.sync_copy(buf_a, out_hbm.at[pl.ds(base + c0, CH)])
            cp_b.wait()
            pltpu.sync_copy(buf_b, out_hbm.at[pl.ds(base + c0 + CH, CH)])
            return carry

        lax.fori_loop(0, rpw // (2 * CH), body, 0)

    return k(table, idxf)


_gather = _gather_sc


def _stats1_body(gx_ref, xyz_ref, Wp1_ref, bp1_ref, acc_ref, p1_ref):
    nb = lax.bitcast_convert_type(gx_ref[:, :, 64:67], jnp.float32)
    ctr = xyz_ref[...][:, None, :]
    rel = (nb - ctr).reshape(CE * K, 3)
    p1 = jnp.dot(rel, Wp1_ref[...], preferred_element_type=jnp.float32) + bp1_ref[...]
    p1_ref[...] = p1

    @pl.when(pl.program_id(0) == 0)
    def _():
        acc_ref[...] = jnp.zeros_like(acc_ref)

    acc_ref[0:1, 0:3] += jnp.sum(p1, axis=0, keepdims=True)
    acc_ref[1:2, 0:3] += jnp.sum(p1 * p1, axis=0, keepdims=True)


def _stats1(G3, xyzf, Wp1, bp1):
    return pl.pallas_call(
        _stats1_body,
        grid=(S // CE,),
        in_specs=[
            pl.BlockSpec((CE, K, 128), lambda i: (i, 0, 1)),
            pl.BlockSpec((CE, 3), lambda i: (i, 0)),
            pl.BlockSpec((3, 3), lambda i: (0, 0)),
            pl.BlockSpec((1, 3), lambda i: (0, 0)),
        ],
        out_specs=[
            pl.BlockSpec((8, 128), lambda i: (0, 0)),
            pl.BlockSpec((CE * K, 3), lambda i: (i, 0)),
        ],
        out_shape=[
            jax.ShapeDtypeStruct((8, 128), jnp.float32),
            jax.ShapeDtypeStruct((NKH, 3), jnp.float32),
        ],
        interpret=_INTERPRET,
    )(G3, xyzf, Wp1, bp1)


def _passE_body(gk_ref, p1_ref, q_ref, sc1_ref, sh1_ref, Wp2f_ref, bp2f_ref,
                w0_ref, acc_ref):
    p1n = jnp.maximum(p1_ref[...] * sc1_ref[...] + sh1_ref[...], 0.0)
    fold = jnp.dot(p1n, Wp2f_ref[...], preferred_element_type=jnp.float32) + bp2f_ref[...]
    kp = gk_ref[:, :, 0:64]
    gk = jnp.concatenate([_unpack_lo(kp), _unpack_hi(kp)], axis=2)
    w0 = gk - q_ref[...][:, None, :] + fold.reshape(CE, K, MID)
    w0_ref[...] = w0
    w2d = w0.reshape(CE * K, MID)

    @pl.when(pl.program_id(0) == 0)
    def _():
        acc_ref[...] = jnp.zeros_like(acc_ref)

    acc_ref[0:1, :] += jnp.sum(w2d, axis=0, keepdims=True)
    acc_ref[1:2, :] += jnp.sum(w2d * w2d, axis=0, keepdims=True)


def _passE(G3, p1, q, sc1, sh1, Wp2f, bp2f):
    return pl.pallas_call(
        _passE_body,
        grid=(S // CE,),
        in_specs=[
            pl.BlockSpec((CE, K, 128), lambda i: (i, 0, 1)),
            pl.BlockSpec((CE * K, 3), lambda i: (i, 0)),
            pl.BlockSpec((CE, MID), lambda i: (i, 0)),
            pl.BlockSpec((1, 3), lambda i: (0, 0)),
            pl.BlockSpec((1, 3), lambda i: (0, 0)),
            pl.BlockSpec((3, MID), lambda i: (0, 0)),
            pl.BlockSpec((1, MID), lambda i: (0, 0)),
        ],
        out_specs=[
            pl.BlockSpec((CE, K, MID), lambda i: (i, 0, 0)),
            pl.BlockSpec((8, 128), lambda i: (0, 0)),
        ],
        out_shape=[
            jax.ShapeDtypeStruct((S, K, MID), jnp.float32),
            jax.ShapeDtypeStruct((8, 128), jnp.float32),
        ],
        interpret=_INTERPRET,
    )(G3, p1, q, sc1, sh1, Wp2f, bp2f)


def _passF_body(w0_ref, sc2_ref, sh2_ref, Ww1_ref, bw1_ref, w1_ref, acc_ref):
    w0 = w0_ref[...].reshape(CE * K, MID)
    w0n = jnp.maximum(w0 * sc2_ref[...] + sh2_ref[...], 0.0)
    w1 = jnp.dot(w0n, Ww1_ref[...], preferred_element_type=jnp.float32) + bw1_ref[...]
    w1_ref[...] = w1.reshape(CE, K, 16)

    @pl.when(pl.program_id(0) == 0)
    def _():
        acc_ref[...] = jnp.zeros_like(acc_ref)

    acc_ref[0:1, 0:16] += jnp.sum(w1, axis=0, keepdims=True)
    acc_ref[1:2, 0:16] += jnp.sum(w1 * w1, axis=0, keepdims=True)


def _passF(w0, sc2, sh2, Ww1, bw1):
    return pl.pallas_call(
        _passF_body,
        grid=(S // CE,),
        in_specs=[
            pl.BlockSpec((CE, K, MID), lambda i: (i, 0, 0)),
            pl.BlockSpec((1, MID), lambda i: (0, 0)),
            pl.BlockSpec((1, MID), lambda i: (0, 0)),
            pl.BlockSpec((MID, 16), lambda i: (0, 0)),
            pl.BlockSpec((1, 16), lambda i: (0, 0)),
        ],
        out_specs=[
            pl.BlockSpec((CE, K, 16), lambda i: (i, 0, 0)),
            pl.BlockSpec((8, 128), lambda i: (0, 0)),
        ],
        out_shape=[
            jax.ShapeDtypeStruct((S, K, 16), jnp.float32),
            jax.ShapeDtypeStruct((8, 128), jnp.float32),
        ],
        interpret=_INTERPRET,
    )(w0, sc2, sh2, Ww1, bw1)


def _passG_body(w1_ref, gv_ref, p1_ref, sc3_ref, sh3_ref, Ww2_ref, bw2_ref,
                sc1_ref, sh1_ref, Wp2_ref, bp2_ref, out_ref):
    w1 = w1_ref[...].reshape(CE * K, 16)
    w1n = jnp.maximum(w1 * sc3_ref[...] + sh3_ref[...], 0.0)
    w2 = (jnp.dot(w1n, Ww2_ref[...], preferred_element_type=jnp.float32)
          + bw2_ref[...]).reshape(CE, K, 32)
    m = jnp.max(w2, axis=1, keepdims=True)
    e = jnp.exp(w2 - m)
    sm = e / jnp.sum(e, axis=1, keepdims=True)
    p1n = jnp.maximum(p1_ref[...] * sc1_ref[...] + sh1_ref[...], 0.0)
    pr = (jnp.dot(p1n, Wp2_ref[...], preferred_element_type=jnp.float32)
          + bp2_ref[...]).reshape(CE, K, OUT)
    vp = gv_ref[...]
    gv = jnp.concatenate([_unpack_lo(vp), _unpack_hi(vp)], axis=2)
    x2 = gv + pr
    wt = jnp.concatenate([sm] * 8, axis=2)
    out_ref[...] = jnp.sum(x2 * wt, axis=1)


def _passG(w1, G3, p1, sc3, sh3, Ww2, bw2, sc1, sh1, Wp2, bp2):
    return pl.pallas_call(
        _passG_body,
        grid=(S // CE,),
        in_specs=[
            pl.BlockSpec((CE, K, 16), lambda i: (i, 0, 0)),
            pl.BlockSpec((CE, K, 128), lambda i: (i, 0, 0)),
            pl.BlockSpec((CE * K, 3), lambda i: (i, 0)),
            pl.BlockSpec((1, 16), lambda i: (0, 0)),
            pl.BlockSpec((1, 16), lambda i: (0, 0)),
            pl.BlockSpec((16, 32), lambda i: (0, 0)),
            pl.BlockSpec((1, 32), lambda i: (0, 0)),
            pl.BlockSpec((1, 3), lambda i: (0, 0)),
            pl.BlockSpec((1, 3), lambda i: (0, 0)),
            pl.BlockSpec((3, OUT), lambda i: (0, 0)),
            pl.BlockSpec((1, OUT), lambda i: (0, 0)),
        ],
        out_specs=pl.BlockSpec((CE, OUT), lambda i: (i, 0)),
        out_shape=jax.ShapeDtypeStruct((S, OUT), jnp.float32),
        interpret=_INTERPRET,
    )(w1, G3, p1, sc3, sh3, Ww2, bw2, sc1, sh1, Wp2, bp2)


def kernel(xyz, points, Wq, bq, Wk, bk, Wv, bv, Wp1, bp1, gp, bp, Wp2, bp2,
           g1, b1, Ww1, bw1, g2, b2, Ww2, bw2):
    ptsf = points.reshape(N, CIN)
    xyzf = xyz.reshape(N, 3)
    Wall = jnp.concatenate([Wq, Wk, Wv], axis=1)
    ball = jnp.concatenate([bq, bk, bv])[None, :]
    q, table = _proj(ptsf, xyzf, Wall, ball)

    G3s = []
    for b in range(B):
        xyz2 = xyz[b]
        idx = _knn1(xyz2, jnp.transpose(xyz2))
        Gb = _gather(table, (idx + b * S).reshape(NKH))
        G3s.append(Gb.reshape(S, K, TWI))

    cnt = jnp.float32(NK)
    accs, p1s = [], []
    for b in range(B):
        a, p1 = _stats1(G3s[b], xyzf[b * S:(b + 1) * S], Wp1, bp1[None, :])
        accs.append(a)
        p1s.append(p1)
    acc1 = accs[0] + accs[1]
    m1 = acc1[0, 0:3] / cnt
    v1 = acc1[1, 0:3] / cnt - m1 * m1
    sc1 = gp * lax.rsqrt(v1 + EPS)
    sh1 = bp - m1 * sc1

    Wp2f = Wp2[:, 0:MID] + Wp2[:, MID:OUT]
    bp2f = (bp2[0:MID] + bp2[MID:OUT])[None, :]
    w0s = []
    acc2 = None
    for b in range(B):
        w0, a = _passE(G3s[b], p1s[b], q[b * S:(b + 1) * S],
                       sc1[None], sh1[None], Wp2f, bp2f)
        w0s.append(w0)
        acc2 = a if acc2 is None else acc2 + a
    m2 = acc2[0] / cnt
    v2 = acc2[1] / cnt - m2 * m2
    sc2 = g1 * lax.rsqrt(v2 + EPS)
    sh2 = b1 - m2 * sc2

    w1s = []
    acc3 = None
    for b in range(B):
        w1, a = _passF(w0s[b], sc2[None], sh2[None], Ww1, bw1[None])
        w1s.append(w1)
        acc3 = a if acc3 is None else acc3 + a
    m3 = acc3[0, 0:16] / cnt
    v3 = acc3[1, 0:16] / cnt - m3 * m3
    sc3 = g2 * lax.rsqrt(v3 + EPS)
    sh3 = b2 - m3 * sc3

    outs = [
        _passG(w1s[b], G3s[b], p1s[b], sc3[None], sh3[None], Ww2, bw2[None],
               sc1[None], sh1[None], Wp2, bp2[None])
        for b in range(B)
    ]
    return jnp.concatenate(outs, axis=0)


# speedup vs baseline: 1.2257x; 1.0134x over previous
import functools

import jax
import jax.numpy as jnp
from jax import lax
from jax.experimental import pallas as pl
from jax.experimental.pallas import tpu as pltpu
from jax.experimental.pallas import tpu_sc as plsc

B = 2
S = 4096
N = B * S
K = 16
CIN = 256
MID = 128
OUT = 256
TWI = 256
NK = N * K
EPS = 1e-5
BIGF = 3.0e38

_INTERPRET = False

RP = 512
RK = 256
CE = 512


def _pack_bf16(a, b):
    ua = lax.bitcast_convert_type(a, jnp.uint32)
    ub = lax.bitcast_convert_type(b, jnp.uint32)
    ra = (ua + jnp.uint32(0x7FFF) + ((ua >> 16) & jnp.uint32(1))) >> 16
    rb = (ub + jnp.uint32(0x7FFF) + ((ub >> 16) & jnp.uint32(1))) >> 16
    return lax.bitcast_convert_type(ra | (rb << 16), jnp.int32)


def _unpack_lo(p):
    return lax.bitcast_convert_type(p << 16, jnp.float32)


def _unpack_hi(p):
    u = lax.bitcast_convert_type(p, jnp.uint32)
    return lax.bitcast_convert_type((u >> 16) << 16, jnp.float32)


def _proj_body(pts_ref, xyz_ref, Wall_ref, ball_ref, q_ref, tab_ref):
    x = pts_ref[...]
    qkv = jnp.dot(x, Wall_ref[...], preferred_element_type=jnp.float32) + ball_ref[...]
    q_ref[...] = qkv[:, 0:MID]
    v = qkv[:, 2 * MID:2 * MID + OUT]
    k = qkv[:, MID:2 * MID]
    tab_ref[:, 0:128] = _pack_bf16(v[:, 0:128], v[:, 128:256])
    kp = _pack_bf16(k[:, 0:64], k[:, 64:128])
    xyzb = lax.bitcast_convert_type(xyz_ref[...], jnp.int32)
    tab_ref[:, 128:256] = jnp.concatenate(
        [kp, xyzb, jnp.zeros((RP, 61), jnp.int32)], axis=1)


def _proj(pts, xyzf, Wall, ball):
    return pl.pallas_call(
        _proj_body,
        grid=(N // RP,),
        in_specs=[
            pl.BlockSpec((RP, CIN), lambda i: (i, 0)),
            pl.BlockSpec((RP, 3), lambda i: (i, 0)),
            pl.BlockSpec((CIN, 2 * MID + OUT), lambda i: (0, 0)),
            pl.BlockSpec((1, 2 * MID + OUT), lambda i: (0, 0)),
        ],
        out_specs=[
            pl.BlockSpec((RP, MID), lambda i: (i, 0)),
            pl.BlockSpec((RP, TWI), lambda i: (i, 0)),
        ],
        out_shape=[
            jax.ShapeDtypeStruct((N, MID), jnp.float32),
            jax.ShapeDtypeStruct((N, TWI), jnp.int32),
        ],
        interpret=_INTERPRET,
    )(pts, xyzf, Wall, ball)


def _knn_body(xyz_ref, xyzT_ref, idx_ref):
    i = pl.program_id(0)
    xr = xyz_ref[...]
    xcT = xyzT_ref[...]
    sqr = jnp.sum(xr * xr, axis=1, keepdims=True)
    sqc = jnp.sum(xcT * xcT, axis=0, keepdims=True)
    d = sqr + sqc - 2.0 * jnp.dot(xr, xcT, preferred_element_type=jnp.float32)
    cols = lax.broadcasted_iota(jnp.int32, (RK, S), 1)
    rows = i * RK + lax.broadcasted_iota(jnp.int32, (RK, 1), 0)
    d = jnp.where(cols == rows, BIGF, d)
    outs = [rows]
    for _ in range(K - 1):
        am = jnp.argmin(d, axis=1).astype(jnp.int32)[:, None]
        outs.append(am)
        d = jnp.where(cols == am, BIGF, d)
    idx_ref[...] = jnp.concatenate(outs, axis=1)


def _knn1(xyz2, xyzT2):
    return pl.pallas_call(
        _knn_body,
        grid=(S // RK,),
        in_specs=[
            pl.BlockSpec((RK, 3), lambda i: (i, 0)),
            pl.BlockSpec((3, S), lambda i: (0, 0)),
        ],
        out_specs=pl.BlockSpec((RK, K), lambda i: (i, 0)),
        out_shape=jax.ShapeDtypeStruct((S, K), jnp.int32),
        interpret=_INTERPRET,
    )(xyz2, xyzT2)


SC_NC, SC_NS = 2, 16
NW = SC_NC * SC_NS
CH = 128
NKH = S * K


def _gather_sc(table, idxf):
    mesh = plsc.VectorSubcoreMesh(core_axis_name="c", subcore_axis_name="s")
    rpw = NKH // NW

    @functools.partial(
        pl.kernel,
        out_type=jax.ShapeDtypeStruct((NKH, TWI), jnp.int32),
        mesh=mesh,
        scratch_types=[
            pltpu.VMEM((rpw,), jnp.int32),
            pltpu.VMEM((CH, TWI), jnp.int32),
            pltpu.VMEM((CH, TWI), jnp.int32),
            pltpu.SemaphoreType.DMA,
            pltpu.SemaphoreType.DMA,
        ],
    )
    def k(table_hbm, idx_hbm, out_hbm, idx_v, buf_a, buf_b, sem_a, sem_b):
        wid = lax.axis_index("s") * SC_NC + lax.axis_index("c")
        base = wid * rpw
        pltpu.sync_copy(idx_hbm.at[pl.ds(base, rpw)], idx_v)

        def body(i, carry):
            c0 = 2 * i * CH
            cp_a = pltpu.async_copy(
                table_hbm.at[idx_v.at[pl.ds(c0, CH)]], buf_a, sem_a)
            cp_b = pltpu.async_copy(
                table_hbm.at[idx_v.at[pl.ds(c0 + CH, CH)]], buf_b, sem_b)
            cp_a.wait()
            pltpu.sync_copy(buf_a, out_hbm.at[pl.ds(base + c0, CH)])
            cp_b.wait()
            pltpu.sync_copy(buf_b, out_hbm.at[pl.ds(base + c0 + CH, CH)])
            return carry

        lax.fori_loop(0, rpw // (2 * CH), body, 0)

    return k(table, idxf)


_gather = _gather_sc


def _stats1_body(gx_ref, xyz_ref, Wp1_ref, bp1_ref, acc_ref, p1_ref):
    nb = lax.bitcast_convert_type(gx_ref[:, :, 64:67], jnp.float32)
    ctr = xyz_ref[...][:, None, :]
    rel = (nb - ctr).reshape(CE * K, 3)
    p1 = jnp.dot(rel, Wp1_ref[...], preferred_element_type=jnp.float32) + bp1_ref[...]
    p1_ref[...] = p1

    @pl.when(pl.program_id(0) == 0)
    def _():
        acc_ref[...] = jnp.zeros_like(acc_ref)

    acc_ref[0:1, 0:3] += jnp.sum(p1, axis=0, keepdims=True)
    acc_ref[1:2, 0:3] += jnp.sum(p1 * p1, axis=0, keepdims=True)


def _stats1(G3, xyzf, Wp1, bp1):
    return pl.pallas_call(
        _stats1_body,
        grid=(S // CE,),
        in_specs=[
            pl.BlockSpec((CE, K, 128), lambda i: (i, 0, 1)),
            pl.BlockSpec((CE, 3), lambda i: (i, 0)),
            pl.BlockSpec((3, 3), lambda i: (0, 0)),
            pl.BlockSpec((1, 3), lambda i: (0, 0)),
        ],
        out_specs=[
            pl.BlockSpec((8, 128), lambda i: (0, 0)),
            pl.BlockSpec((CE * K, 3), lambda i: (i, 0)),
        ],
        out_shape=[
            jax.ShapeDtypeStruct((8, 128), jnp.float32),
            jax.ShapeDtypeStruct((NKH, 3), jnp.float32),
        ],
        interpret=_INTERPRET,
    )(G3, xyzf, Wp1, bp1)


def _passE_body(gk_ref, p1_ref, q_ref, sc1_ref, sh1_ref, Wp2f_ref, bp2f_ref,
                w0_ref, acc_ref):
    p1n = jnp.maximum(p1_ref[...] * sc1_ref[...] + sh1_ref[...], 0.0)
    fold = jnp.dot(p1n, Wp2f_ref[...], preferred_element_type=jnp.float32) + bp2f_ref[...]
    kp = gk_ref[:, :, 0:64]
    gk = jnp.concatenate([_unpack_lo(kp), _unpack_hi(kp)], axis=2)
    w0 = gk - q_ref[...][:, None, :] + fold.reshape(CE, K, MID)
    w0_ref[...] = w0
    w2d = w0.reshape(CE * K, MID)

    @pl.when(pl.program_id(0) == 0)
    def _():
        acc_ref[...] = jnp.zeros_like(acc_ref)

    acc_ref[0:1, :] += jnp.sum(w2d, axis=0, keepdims=True)
    acc_ref[1:2, :] += jnp.sum(w2d * w2d, axis=0, keepdims=True)


def _passE(G3, p1, q, sc1, sh1, Wp2f, bp2f):
    return pl.pallas_call(
        _passE_body,
        grid=(S // CE,),
        in_specs=[
            pl.BlockSpec((CE, K, 128), lambda i: (i, 0, 1)),
            pl.BlockSpec((CE * K, 3), lambda i: (i, 0)),
            pl.BlockSpec((CE, MID), lambda i: (i, 0)),
            pl.BlockSpec((1, 3), lambda i: (0, 0)),
            pl.BlockSpec((1, 3), lambda i: (0, 0)),
            pl.BlockSpec((3, MID), lambda i: (0, 0)),
            pl.BlockSpec((1, MID), lambda i: (0, 0)),
        ],
        out_specs=[
            pl.BlockSpec((CE, K, MID), lambda i: (i, 0, 0)),
            pl.BlockSpec((8, 128), lambda i: (0, 0)),
        ],
        out_shape=[
            jax.ShapeDtypeStruct((S, K, MID), jnp.float32),
            jax.ShapeDtypeStruct((8, 128), jnp.float32),
        ],
        interpret=_INTERPRET,
    )(G3, p1, q, sc1, sh1, Wp2f, bp2f)


def _passF_body(w0_ref, sc2_ref, sh2_ref, Ww1_ref, bw1_ref, w1_ref, acc_ref):
    w0 = w0_ref[...].reshape(CE * K, MID)
    w0n = jnp.maximum(w0 * sc2_ref[...] + sh2_ref[...], 0.0)
    w1 = jnp.dot(w0n, Ww1_ref[...], preferred_element_type=jnp.float32) + bw1_ref[...]
    w1_ref[...] = w1.reshape(CE, K, 16)

    @pl.when(pl.program_id(0) == 0)
    def _():
        acc_ref[...] = jnp.zeros_like(acc_ref)

    acc_ref[0:1, 0:16] += jnp.sum(w1, axis=0, keepdims=True)
    acc_ref[1:2, 0:16] += jnp.sum(w1 * w1, axis=0, keepdims=True)


def _passF(w0, sc2, sh2, Ww1, bw1):
    return pl.pallas_call(
        _passF_body,
        grid=(S // CE,),
        in_specs=[
            pl.BlockSpec((CE, K, MID), lambda i: (i, 0, 0)),
            pl.BlockSpec((1, MID), lambda i: (0, 0)),
            pl.BlockSpec((1, MID), lambda i: (0, 0)),
            pl.BlockSpec((MID, 16), lambda i: (0, 0)),
            pl.BlockSpec((1, 16), lambda i: (0, 0)),
        ],
        out_specs=[
            pl.BlockSpec((CE, K, 16), lambda i: (i, 0, 0)),
            pl.BlockSpec((8, 128), lambda i: (0, 0)),
        ],
        out_shape=[
            jax.ShapeDtypeStruct((S, K, 16), jnp.float32),
            jax.ShapeDtypeStruct((8, 128), jnp.float32),
        ],
        interpret=_INTERPRET,
    )(w0, sc2, sh2, Ww1, bw1)


def _passG_body(w1_ref, gv_ref, p1_ref, sc3_ref, sh3_ref, Ww2_ref, bw2_ref,
                sc1_ref, sh1_ref, Wp2_ref, bp2_ref, out_ref):
    w1 = w1_ref[...].reshape(CE * K, 16)
    w1n = jnp.maximum(w1 * sc3_ref[...] + sh3_ref[...], 0.0)
    w2 = (jnp.dot(w1n, Ww2_ref[...], preferred_element_type=jnp.float32)
          + bw2_ref[...]).reshape(CE, K, 32)
    m = jnp.max(w2, axis=1, keepdims=True)
    e = jnp.exp(w2 - m)
    sm = e / jnp.sum(e, axis=1, keepdims=True)
    p1n = jnp.maximum(p1_ref[...] * sc1_ref[...] + sh1_ref[...], 0.0)
    pr = (jnp.dot(p1n, Wp2_ref[...], preferred_element_type=jnp.float32)
          + bp2_ref[...]).reshape(CE, K, OUT)
    vp = gv_ref[...]
    gv = jnp.concatenate([_unpack_lo(vp), _unpack_hi(vp)], axis=2)
    x2 = gv + pr
    wt = jnp.concatenate([sm] * 8, axis=2)
    out_ref[...] = jnp.sum(x2 * wt, axis=1)


def _passG(w1, G3, p1, sc3, sh3, Ww2, bw2, sc1, sh1, Wp2, bp2):
    return pl.pallas_call(
        _passG_body,
        grid=(S // CE,),
        in_specs=[
            pl.BlockSpec((CE, K, 16), lambda i: (i, 0, 0)),
            pl.BlockSpec((CE, K, 128), lambda i: (i, 0, 0)),
            pl.BlockSpec((CE * K, 3), lambda i: (i, 0)),
            pl.BlockSpec((1, 16), lambda i: (0, 0)),
            pl.BlockSpec((1, 16), lambda i: (0, 0)),
            pl.BlockSpec((16, 32), lambda i: (0, 0)),
            pl.BlockSpec((1, 32), lambda i: (0, 0)),
            pl.BlockSpec((1, 3), lambda i: (0, 0)),
            pl.BlockSpec((1, 3), lambda i: (0, 0)),
            pl.BlockSpec((3, OUT), lambda i: (0, 0)),
            pl.BlockSpec((1, OUT), lambda i: (0, 0)),
        ],
        out_specs=pl.BlockSpec((CE, OUT), lambda i: (i, 0)),
        out_shape=jax.ShapeDtypeStruct((S, OUT), jnp.float32),
        interpret=_INTERPRET,
    )(w1, G3, p1, sc3, sh3, Ww2, bw2, sc1, sh1, Wp2, bp2)


def kernel(xyz, points, Wq, bq, Wk, bk, Wv, bv, Wp1, bp1, gp, bp, Wp2, bp2,
           g1, b1, Ww1, bw1, g2, b2, Ww2, bw2):
    ptsf = points.reshape(N, CIN)
    xyzf = xyz.reshape(N, 3)
    Wall = jnp.concatenate([Wq, Wk, Wv], axis=1)
    ball = jnp.concatenate([bq, bk, bv])[None, :]
    q, table = _proj(ptsf, xyzf, Wall, ball)

    G3s = []
    for b in range(B):
        xyz2 = xyz[b]
        idx = _knn1(xyz2, jnp.transpose(xyz2))
        Gb = _gather(table, (idx + b * S).reshape(NKH))
        G3s.append(Gb.reshape(S, K, TWI))

    cnt = jnp.float32(NK)
    accs, p1s = [], []
    for b in range(B):
        a, p1 = _stats1(G3s[b], xyzf[b * S:(b + 1) * S], Wp1, bp1[None, :])
        accs.append(a)
        p1s.append(p1)
    acc1 = accs[0] + accs[1]
    m1 = acc1[0, 0:3] / cnt
    v1 = acc1[1, 0:3] / cnt - m1 * m1
    sc1 = gp * lax.rsqrt(v1 + EPS)
    sh1 = bp - m1 * sc1

    Wp2f = Wp2[:, 0:MID] + Wp2[:, MID:OUT]
    bp2f = (bp2[0:MID] + bp2[MID:OUT])[None, :]
    w0s = []
    acc2 = None
    for b in range(B):
        w0, a = _passE(G3s[b], p1s[b], q[b * S:(b + 1) * S],
                       sc1[None], sh1[None], Wp2f, bp2f)
        w0s.append(w0)
        acc2 = a if acc2 is None else acc2 + a
    m2 = acc2[0] / cnt
    v2 = acc2[1] / cnt - m2 * m2
    sc2 = g1 * lax.rsqrt(v2 + EPS)
    sh2 = b1 - m2 * sc2

    w1s = []
    acc3 = None
    for b in range(B):
        w1, a = _passF(w0s[b], sc2[None], sh2[None], Ww1, bw1[None])
        w1s.append(w1)
        acc3 = a if acc3 is None else acc3 + a
    m3 = acc3[0, 0:16] / cnt
    v3 = acc3[1, 0:16] / cnt - m3 * m3
    sc3 = g2 * lax.rsqrt(v3 + EPS)
    sh3 = b2 - m3 * sc3

    outs = [
        _passG(w1s[b], G3s[b], p1s[b], sc3[None], sh3[None], Ww2, bw2[None],
               sc1[None], sh1[None], Wp2, bp2[None])
        for b in range(B)
    ]
    return jnp.concatenate(outs, axis=0)
